# baseline (speedup 1.0000x reference)
import functools
import math

import jax
import jax.numpy as jnp
from jax import lax
from jax.experimental import pallas as pl
from jax.experimental.pallas import tpu as pltpu
from jax.experimental.pallas import tpu_sc as plsc

B, N, M, Q, D, K, C = 4, 8192, 2048, 2048, 32, 16, 128
NG = N // 16
ROWS = B * Q


def _leaky(x):
    return jnp.where(x >= 0, x, 0.2 * x)


def _k1_body(gpc_ref, wg1_ref, bg1_ref, wg2_ref, bg2_ref,
             pf_ref, gmax_ref, planar_ref):
    j = pl.program_id(1)
    g = gpc_ref[0]
    gp = jnp.concatenate([g, jnp.zeros((g.shape[0], 1), jnp.float32)], axis=1)
    pf = _leaky(jax.lax.dot_general(gp, wg1_ref[...],
                                    (((1,), (0,)), ((), ())),
                                    preferred_element_type=jnp.float32)
                + bg1_ref[...][None, :])
    pf_ref[0] = pf
    gg = _leaky(jax.lax.dot_general(pf, wg2_ref[...],
                                    (((1,), (0,)), ((), ())),
                                    preferred_element_type=jnp.float32)
                + bg2_ref[...][None, :])
    part = jnp.max(gg, axis=0, keepdims=True)

    @pl.when(j == 0)
    def _():
        gmax_ref[...] = jnp.full_like(gmax_ref, -jnp.inf)

    gmax_ref[0] = jnp.maximum(gmax_ref[0], part)
    planar_ref[0] = jnp.transpose(gp, (1, 0))


def _run_k1(global_pc, Wg1p, bg1, Wg2, bg2):
    return pl.pallas_call(
        _k1_body,
        grid=(B, N // 1024),
        in_specs=[
            pl.BlockSpec((1, 1024, 3), lambda b, j: (b, j, 0)),
            pl.BlockSpec((4, 128), lambda b, j: (0, 0)),
            pl.BlockSpec((128,), lambda b, j: (0,)),
            pl.BlockSpec((128, 256), lambda b, j: (0, 0)),
            pl.BlockSpec((256,), lambda b, j: (0,)),
        ],
        out_specs=[
            pl.BlockSpec((1, 1024, 128), lambda b, j: (b, j, 0)),
            pl.BlockSpec((1, 1, 256), lambda b, j: (b, 0, 0)),
            pl.BlockSpec((1, 4, 1024), lambda b, j: (b, 0, j)),
        ],
        out_shape=[
            jax.ShapeDtypeStruct((B, N, 128), jnp.float32),
            jax.ShapeDtypeStruct((B, 1, 256), jnp.float32),
            jax.ShapeDtypeStruct((B, 4, N), jnp.float32),
        ],
    )(global_pc, Wg1p, bg1, Wg2, bg2)


def _k2_body(q_ref, g_ref, gmin_ref):
    q = q_ref[0]
    g = g_ref[0]
    qp = jnp.concatenate([q, jnp.zeros((q.shape[0], 1), jnp.float32)], axis=1)
    gp = jnp.concatenate([g, jnp.zeros((g.shape[0], 1), jnp.float32)], axis=1)
    qq = jnp.sum(q * q, axis=1, keepdims=True)
    gg = jnp.sum(g * g, axis=1)[None, :]
    cross = jax.lax.dot_general(qp, gp, (((1,), (1,)), ((), ())),
                                preferred_element_type=jnp.float32)
    d = qq - 2.0 * cross + gg
    gmin_ref[0] = jnp.min(d.reshape(512, 128, 16), axis=-1)


def _run_k2(query, global_pc):
    return pl.pallas_call(
        _k2_body,
        grid=(B, Q // 512, N // 2048),
        in_specs=[
            pl.BlockSpec((1, 512, 3), lambda b, i, j: (b, i, 0)),
            pl.BlockSpec((1, 2048, 3), lambda b, i, j: (b, j, 0)),
        ],
        out_specs=pl.BlockSpec((1, 512, 128), lambda b, i, j: (b, i, j)),
        out_shape=jax.ShapeDtypeStruct((B, Q, NG), jnp.float32),
    )(query, global_pc)


def _k3_body(lpc_ref, vox_ref, q_ref,
             wl1_ref, bl1_ref, wl2_ref, bl2_ref,
             wvox_ref, bvox_ref, wp_ref, bp_ref, gamma_ref, beta_ref,
             lc_ref, vf_ref, pe_ref, qplanar_ref):
    lp = lpc_ref[...].reshape(B * M, 3)
    lp4 = jnp.concatenate([lp, jnp.zeros((B * M, 1), jnp.float32)], axis=1)
    lh = _leaky(jax.lax.dot_general(lp4, wl1_ref[...], (((1,), (0,)), ((), ())),
                                    preferred_element_type=jnp.float32)
                + bl1_ref[...][None, :])
    l2 = _leaky(jax.lax.dot_general(lh, wl2_ref[...], (((1,), (0,)), ((), ())),
                                    preferred_element_type=jnp.float32)
                + bl2_ref[...][None, :])
    lc_ref[...] = jnp.max(l2.reshape(B, M, 256), axis=1, keepdims=True)

    w2d = wvox_ref[...]
    for b in range(B):
        conv = jax.lax.dot_general(w2d, vox_ref[b], (((0,), (0,)), ((), ())),
                                   preferred_element_type=jnp.float32)
        conv = _leaky(conv + bvox_ref[...][:, None])
        vf_ref[b, 0] = jnp.mean(conv, axis=1)

    q = q_ref[...].reshape(B * Q, 3)
    q4 = jnp.concatenate([q, jnp.zeros((B * Q, 1), jnp.float32)], axis=1)
    pe_raw = jax.lax.dot_general(q4, wp_ref[...], (((1,), (0,)), ((), ())),
                                 preferred_element_type=jnp.float32) \
        + bp_ref[...][None, :]
    qbar = jnp.mean(q4, axis=0, keepdims=True)
    second = jax.lax.dot_general(q4, q4, (((0,), (0,)), ((), ())),
                                 preferred_element_type=jnp.float32) / (B * Q)
    cov = second - jax.lax.dot_general(qbar, qbar, (((0,), (0,)), ((), ())),
                                       preferred_element_type=jnp.float32)
    wp = wp_ref[...]
    wc = jax.lax.dot_general(cov, wp, (((1,), (0,)), ((), ())),
                             preferred_element_type=jnp.float32)
    var = jnp.sum(wp * wc, axis=0)
    mu = jax.lax.dot_general(qbar, wp, (((1,), (0,)), ((), ())),
                             preferred_element_type=jnp.float32)[0] + bp_ref[...]
    inv = gamma_ref[...] / jnp.sqrt(var + 1e-5)
    pe = _leaky((pe_raw - mu[None, :]) * inv[None, :] + beta_ref[...][None, :])
    pe_ref[...] = pe.reshape(B, Q, 128)
    qplanar_ref[...] = jnp.transpose(q4.reshape(B, Q, 4), (0, 2, 1))


def _voxel_patches(voxel):
    v = voxel[..., 0]
    vp = jnp.pad(v, ((0, 0), (0, 2), (0, 2), (0, 2)))
    cols = []
    for i in range(3):
        for j in range(3):
            for k in range(3):
                c = lax.slice(vp, (0, i, j, k), (B, i + 32, j + 32, k + 32),
                              (1, 2, 2, 2))
                cols.append(c.reshape(B, 1, 4096))
    cols.append(jnp.zeros((B, 5, 4096), jnp.float32))
    return jnp.concatenate(cols, axis=1)


def _run_k3(local_pc, voxel, query, Wl1p, bl1, Wl2, bl2, Wvox2d, bvox,
            Wpp, bp, gamma, beta):
    voxel = _voxel_patches(voxel)
    return pl.pallas_call(
        _k3_body,
        out_shape=[
            jax.ShapeDtypeStruct((B, 1, 256), jnp.float32),
            jax.ShapeDtypeStruct((B, 1, 64), jnp.float32),
            jax.ShapeDtypeStruct((B, Q, 128), jnp.float32),
            jax.ShapeDtypeStruct((B, 4, Q), jnp.float32),
        ],
    )(local_pc, voxel, query, Wl1p, bl1, Wl2, bl2, Wvox2d, bvox,
      Wpp, bp, gamma, beta)


def _k5_body(kf_ref, pos_ref, pe_ref, gc_ref, lc_ref, vf_ref,
             wpos_ref, bpos_ref, wq_ref, wk_ref, wv_ref,
             wgc_ref, wlc_ref, wqf_ref, wpe_ref, wvf_ref, bd1_ref,
             wd2_ref, bd2_ref, out_ref):
    kf = kf_ref[...].reshape(256 * K, 128)
    pos = pos_ref[...].reshape(256 * K, 4)
    pos_enc = _leaky(jax.lax.dot_general(pos, wpos_ref[...],
                                         (((1,), (0,)), ((), ())),
                                         preferred_element_type=jnp.float32)
                     + bpos_ref[...][None, :])
    fq = jax.lax.dot_general(kf, wq_ref[...], (((1,), (0,)), ((), ())),
                             preferred_element_type=jnp.float32)
    fk = jax.lax.dot_general(pos_enc, wk_ref[...], (((1,), (0,)), ((), ())),
                             preferred_element_type=jnp.float32)
    fv = jax.lax.dot_general(kf + pos_enc, wv_ref[...], (((1,), (0,)), ((), ())),
                             preferred_element_type=jnp.float32)
    logits = jnp.sum(fq * fk, axis=1).reshape(256, K) / math.sqrt(float(C))
    mx = jnp.max(logits, axis=1, keepdims=True)
    e = jnp.exp(logits - mx)
    attn = e / jnp.sum(e, axis=1, keepdims=True)
    qf = jnp.sum(attn[:, :, None] * fv.reshape(256, K, 128), axis=1)

    cbase = (jax.lax.dot_general(gc_ref[0], wgc_ref[...],
                                 (((1,), (0,)), ((), ())),
                                 preferred_element_type=jnp.float32)
             + jax.lax.dot_general(lc_ref[0], wlc_ref[...],
                                   (((1,), (0,)), ((), ())),
                                   preferred_element_type=jnp.float32)
             + jax.lax.dot_general(vf_ref[0], wvf_ref[...],
                                   (((1,), (0,)), ((), ())),
                                   preferred_element_type=jnp.float32))
    pe = pe_ref[0]
    h = _leaky(jax.lax.dot_general(qf, wqf_ref[...], (((1,), (0,)), ((), ())),
                                   preferred_element_type=jnp.float32)
               + jax.lax.dot_general(pe, wpe_ref[...], (((1,), (0,)), ((), ())),
                                     preferred_element_type=jnp.float32)
               + cbase + bd1_ref[...][None, :])
    o = jax.lax.dot_general(h, wd2_ref[...], (((1,), (0,)), ((), ())),
                            preferred_element_type=jnp.float32) + bd2_ref[...]
    out_ref[0, 0] = o[:, 0]


def _run_k5(knn_feat, knn_pos, pe, gmax, lc, vf,
            WposT, bpos, WqT, WkT, WvT, Wgc, Wlc, Wqf, Wpe, Wvf, bd1,
            Wd2T, bd2):
    nt = Q // 256
    return pl.pallas_call(
        _k5_body,
        grid=(B, nt),
        in_specs=[
            pl.BlockSpec((256, K, 128), lambda b, t: (b * nt + t, 0, 0)),
            pl.BlockSpec((256, K, 4), lambda b, t: (b * nt + t, 0, 0)),
            pl.BlockSpec((1, 256, 128), lambda b, t: (b, t, 0)),
            pl.BlockSpec((1, 1, 256), lambda b, t: (b, 0, 0)),
            pl.BlockSpec((1, 1, 256), lambda b, t: (b, 0, 0)),
            pl.BlockSpec((1, 1, 64), lambda b, t: (b, 0, 0)),
            pl.BlockSpec((4, 128), lambda b, t: (0, 0)),
            pl.BlockSpec((128,), lambda b, t: (0,)),
            pl.BlockSpec((128, 128), lambda b, t: (0, 0)),
            pl.BlockSpec((128, 128), lambda b, t: (0, 0)),
            pl.BlockSpec((128, 128), lambda b, t: (0, 0)),
            pl.BlockSpec((256, 256), lambda b, t: (0, 0)),
            pl.BlockSpec((256, 256), lambda b, t: (0, 0)),
            pl.BlockSpec((128, 256), lambda b, t: (0, 0)),
            pl.BlockSpec((128, 256), lambda b, t: (0, 0)),
            pl.BlockSpec((64, 256), lambda b, t: (0, 0)),
            pl.BlockSpec((256,), lambda b, t: (0,)),
            pl.BlockSpec((256, 1), lambda b, t: (0, 0)),
            pl.BlockSpec((1,), lambda b, t: (0,)),
        ],
        out_specs=pl.BlockSpec((1, 1, 256), lambda b, t: (b, 0, t)),
        out_shape=jax.ShapeDtypeStruct((B, 1, Q), jnp.float32),
    )(knn_feat, knn_pos, pe, gmax, lc, vf,
      WposT, bpos, WqT, WkT, WvT, Wgc, Wlc, Wqf, Wpe, Wvf, bd1, Wd2T, bd2)


_THR_PAD = 2e-3
RPW = ROWS // 32


def _k4_body(gmin_hbm, qpl_hbm, planar_hbm, pf_hbm, kf_out, pos_out,
             xs, ys, zs, qbuf, gmin_row, gidbuf, dbuf, ibuf,
             idx16, featbuf, posbuf, sem):
    nc = 2
    wid = lax.axis_index("s") * nc + lax.axis_index("c")
    base = wid * RPW
    b = base // Q
    qoff = base % Q
    iota = lax.iota(jnp.int32, 16)
    inf = jnp.float32(jnp.inf)

    pltpu.sync_copy(planar_hbm.at[b, 0], xs)
    pltpu.sync_copy(planar_hbm.at[b, 1], ys)
    pltpu.sync_copy(planar_hbm.at[b, 2], zs)
    pltpu.sync_copy(qpl_hbm.at[b, :, pl.ds(qoff, RPW)], qbuf)
    plsc.store_scatter(posbuf, [iota, jnp.full((16,), 3, jnp.int32)],
                       jnp.zeros((16,), jnp.float32))

    def row_body(i, _):
        r = base + i
        pltpu.sync_copy(gmin_hbm.at[r], gmin_row)
        qx, qy, qz = qbuf[0, i], qbuf[1, i], qbuf[2, i]

        def pa(j, acc):
            return jnp.minimum(acc, gmin_row[pl.ds(j * 16, 16)])
        acc = lax.fori_loop(0, NG // 16, pa, jnp.full((16,), inf, jnp.float32))
        thr = jnp.max(acc) + _THR_PAD

        def pb(j, cnt):
            v = gmin_row[pl.ds(j * 16, 16)]
            m = v <= thr
            plsc.store_compressed(gidbuf.at[pl.ds(cnt, 16)], iota + j * 16, m)
            return cnt + jnp.sum(m.astype(jnp.int32))
        cnt = lax.fori_loop(0, NG // 16, pb, 0)

        def pc(ci, cnt2):
            gid = gidbuf[ci]
            idxv = gid * 16 + iota
            px = plsc.load_gather(xs, [idxv])
            py = plsc.load_gather(ys, [idxv])
            pz = plsc.load_gather(zs, [idxv])
            dx, dy, dz = px - qx, py - qy, pz - qz
            d2 = dx * dx + dy * dy + dz * dz
            m = d2 <= thr
            plsc.store_compressed(dbuf.at[pl.ds(cnt2, 16)], d2, m)
            plsc.store_compressed(ibuf.at[pl.ds(cnt2, 16)], idxv, m)
            return cnt2 + jnp.sum(m.astype(jnp.int32))
        cnt2 = lax.fori_loop(0, cnt, pc, 0)

        def pd(ch, ti):
            tv, tidx = ti
            off = ch * 16
            dv = dbuf[pl.ds(off, 16)]
            iv = ibuf[pl.ds(off, 16)]
            dv = jnp.where(iota < (cnt2 - off), dv, inf)
            sv, si = plsc.sort_key_val(dv, iv)
            rv, ri = lax.rev(sv, (0,)), lax.rev(si, (0,))
            keep = tv <= rv
            wv = jnp.where(keep, tv, rv)
            wi = jnp.where(keep, tidx, ri)
            return plsc.sort_key_val(wv, wi)
        tv0 = jnp.full((16,), inf, jnp.float32)
        ti0 = jnp.zeros((16,), jnp.int32)
        _, topi = lax.fori_loop(0, (cnt2 + 15) // 16, pd, (tv0, ti0))

        px = plsc.load_gather(xs, [topi])
        py = plsc.load_gather(ys, [topi])
        pz = plsc.load_gather(zs, [topi])
        plsc.store_scatter(posbuf, [iota, jnp.zeros((16,), jnp.int32)], qx - px)
        plsc.store_scatter(posbuf, [iota, jnp.full((16,), 1, jnp.int32)], qy - py)
        plsc.store_scatter(posbuf, [iota, jnp.full((16,), 2, jnp.int32)], qz - pz)
        pltpu.sync_copy(posbuf, pos_out.at[r])
        idx16[...] = topi + b * N
        pltpu.async_copy(pf_hbm.at[idx16], featbuf, sem).wait()
        pltpu.sync_copy(featbuf, kf_out.at[r])
        return 0

    lax.fori_loop(0, RPW, row_body, 0)


def _run_k4(gmin, qplanar, planar, pf_flat):
    f32, i32 = jnp.float32, jnp.int32
    mesh = plsc.VectorSubcoreMesh(core_axis_name="c", subcore_axis_name="s")
    kfn = functools.partial(
        pl.kernel,
        mesh=mesh,
        out_type=[
            jax.ShapeDtypeStruct((ROWS, K, 128), f32),
            jax.ShapeDtypeStruct((ROWS, K, 4), f32),
        ],
        scratch_types=[
            pltpu.VMEM((N,), f32), pltpu.VMEM((N,), f32), pltpu.VMEM((N,), f32),
            pltpu.VMEM((4, RPW), f32),
            pltpu.VMEM((NG,), f32),
            pltpu.VMEM((NG + 16,), i32),
            pltpu.VMEM((N + 16,), f32),
            pltpu.VMEM((N + 16,), i32),
            pltpu.VMEM((16,), i32),
            pltpu.VMEM((16, 128), f32),
            pltpu.VMEM((16, 4), f32),
            pltpu.SemaphoreType.DMA,
        ],
    )(_k4_body)
    return kfn(gmin.reshape(ROWS, NG), qplanar, planar, pf_flat)


def _knn_placeholder(gmin, query, global_pc, pf):
    d = (jnp.sum(query * query, axis=-1)[..., None]
         - 2.0 * jnp.einsum('bqd,bnd->bqn', query, global_pc)
         + jnp.sum(global_pc * global_pc, axis=-1)[:, None, :])
    _, idx = jax.lax.top_k(-d, K)
    knn_xyz = jax.vmap(lambda p, i: p[i])(global_pc, idx)
    knn_feat = jax.vmap(lambda f, i: f[i])(pf, idx)
    knn_pos = query[:, :, None, :] - knn_xyz
    knn_pos = jnp.concatenate(
        [knn_pos, jnp.zeros(knn_pos.shape[:-1] + (1,), jnp.float32)], axis=-1)
    return (knn_feat.reshape(ROWS, K, 128), knn_pos.reshape(ROWS, K, 4))


def kernel(global_pc, local_pc, query, voxel, Wg1, bg1, Wg2, bg2,
           Wl1, bl1, Wl2, bl2, Wvox, bvox, Wp, bp, gamma, beta,
           Wpos, bpos, Wq, Wk, Wv, Wd1, bd1, Wd2, bd2):
    f32 = jnp.float32
    pad1 = lambda w: jnp.concatenate([w, jnp.zeros((1,) + w.shape[1:], f32)], 0)
    Wg1p = pad1(Wg1)
    Wl1p = pad1(Wl1)
    Wvox2d = jnp.concatenate(
        [Wvox.reshape(64, 27).T, jnp.zeros((5, 64), f32)], axis=0)
    WpT = Wp.T
    Wpp = pad1(WpT)
    WposT = jnp.concatenate([Wpos.T, jnp.zeros((1, 128), f32)], 0)

    pf, gmax, planar = _run_k1(global_pc, Wg1p, bg1, Wg2, bg2)
    gmin = _run_k2(query, global_pc)
    lc, vf, pe, qplanar = _run_k3(local_pc, voxel, query, Wl1p, bl1, Wl2, bl2,
                                  Wvox2d, bvox, Wpp, bp, gamma, beta)

    knn_feat, knn_pos = _knn_placeholder(gmin, query, global_pc, pf)

    out = _run_k5(knn_feat, knn_pos, pe, gmax, lc, vf,
                  WposT, bpos, Wq.T, Wk.T, Wv.T,
                  Wd1[:, 0:256].T, Wd1[:, 256:512].T, Wd1[:, 512:640].T,
                  Wd1[:, 640:768].T, Wd1[:, 768:832].T, bd1,
                  Wd2.T, bd2)
    return out


# speedup vs baseline: 8.5618x; 8.5618x over previous
import functools
import math

import jax
import jax.numpy as jnp
from jax import lax
from jax.experimental import pallas as pl
from jax.experimental.pallas import tpu as pltpu
from jax.experimental.pallas import tpu_sc as plsc

B, N, M, Q, D, K, C = 4, 8192, 2048, 2048, 32, 16, 128
NG = N // 16
ROWS = B * Q


def _leaky(x):
    return jnp.where(x >= 0, x, 0.2 * x)


def _k1_body(gpc_ref, wg1_ref, bg1_ref, wg2_ref, bg2_ref,
             pf_ref, gmax_ref, planar_ref):
    j = pl.program_id(1)
    g = gpc_ref[0]
    gp = jnp.concatenate([g, jnp.zeros((g.shape[0], 1), jnp.float32)], axis=1)
    pf = _leaky(jax.lax.dot_general(gp, wg1_ref[...],
                                    (((1,), (0,)), ((), ())),
                                    preferred_element_type=jnp.float32)
                + bg1_ref[...][None, :])
    pf_ref[0] = pf
    gg = _leaky(jax.lax.dot_general(pf, wg2_ref[...],
                                    (((1,), (0,)), ((), ())),
                                    preferred_element_type=jnp.float32)
                + bg2_ref[...][None, :])
    part = jnp.max(gg, axis=0, keepdims=True)

    @pl.when(j == 0)
    def _():
        gmax_ref[...] = jnp.full_like(gmax_ref, -jnp.inf)

    gmax_ref[0] = jnp.maximum(gmax_ref[0], part)
    planar_ref[0] = jnp.transpose(gp, (1, 0))


def _run_k1(global_pc, Wg1p, bg1, Wg2, bg2):
    return pl.pallas_call(
        _k1_body,
        grid=(B, N // 1024),
        in_specs=[
            pl.BlockSpec((1, 1024, 3), lambda b, j: (b, j, 0)),
            pl.BlockSpec((4, 128), lambda b, j: (0, 0)),
            pl.BlockSpec((128,), lambda b, j: (0,)),
            pl.BlockSpec((128, 256), lambda b, j: (0, 0)),
            pl.BlockSpec((256,), lambda b, j: (0,)),
        ],
        out_specs=[
            pl.BlockSpec((1, 1024, 128), lambda b, j: (b, j, 0)),
            pl.BlockSpec((1, 1, 256), lambda b, j: (b, 0, 0)),
            pl.BlockSpec((1, 4, 1024), lambda b, j: (b, 0, j)),
        ],
        out_shape=[
            jax.ShapeDtypeStruct((B, N, 128), jnp.float32),
            jax.ShapeDtypeStruct((B, 1, 256), jnp.float32),
            jax.ShapeDtypeStruct((B, 4, N), jnp.float32),
        ],
    )(global_pc, Wg1p, bg1, Wg2, bg2)


def _k2_body(q_ref, g_ref, gmin_ref):
    q = q_ref[0]
    g = g_ref[0]
    qp = jnp.concatenate([q, jnp.zeros((q.shape[0], 1), jnp.float32)], axis=1)
    gp = jnp.concatenate([g, jnp.zeros((g.shape[0], 1), jnp.float32)], axis=1)
    qq = jnp.sum(q * q, axis=1, keepdims=True)
    gg = jnp.sum(g * g, axis=1)[None, :]
    cross = jax.lax.dot_general(qp, gp, (((1,), (1,)), ((), ())),
                                preferred_element_type=jnp.float32)
    d = qq - 2.0 * cross + gg
    gmin_ref[0] = jnp.min(d.reshape(512, 128, 16), axis=-1)


def _run_k2(query, global_pc):
    return pl.pallas_call(
        _k2_body,
        grid=(B, Q // 512, N // 2048),
        in_specs=[
            pl.BlockSpec((1, 512, 3), lambda b, i, j: (b, i, 0)),
            pl.BlockSpec((1, 2048, 3), lambda b, i, j: (b, j, 0)),
        ],
        out_specs=pl.BlockSpec((1, 512, 128), lambda b, i, j: (b, i, j)),
        out_shape=jax.ShapeDtypeStruct((B, Q, NG), jnp.float32),
    )(query, global_pc)


def _k3_body(lpc_ref, vox_ref, q_ref,
             wl1_ref, bl1_ref, wl2_ref, bl2_ref,
             wvox_ref, bvox_ref, wp_ref, bp_ref, gamma_ref, beta_ref,
             lc_ref, vf_ref, pe_ref, qplanar_ref):
    lp = lpc_ref[...].reshape(B * M, 3)
    lp4 = jnp.concatenate([lp, jnp.zeros((B * M, 1), jnp.float32)], axis=1)
    lh = _leaky(jax.lax.dot_general(lp4, wl1_ref[...], (((1,), (0,)), ((), ())),
                                    preferred_element_type=jnp.float32)
                + bl1_ref[...][None, :])
    l2 = _leaky(jax.lax.dot_general(lh, wl2_ref[...], (((1,), (0,)), ((), ())),
                                    preferred_element_type=jnp.float32)
                + bl2_ref[...][None, :])
    lc_ref[...] = jnp.max(l2.reshape(B, M, 256), axis=1, keepdims=True)

    w2d = wvox_ref[...]
    for b in range(B):
        conv = jax.lax.dot_general(w2d, vox_ref[b], (((0,), (0,)), ((), ())),
                                   preferred_element_type=jnp.float32)
        conv = _leaky(conv + bvox_ref[...][:, None])
        vf_ref[b, 0] = jnp.mean(conv, axis=1)

    q = q_ref[...].reshape(B * Q, 3)
    q4 = jnp.concatenate([q, jnp.zeros((B * Q, 1), jnp.float32)], axis=1)
    pe_raw = jax.lax.dot_general(q4, wp_ref[...], (((1,), (0,)), ((), ())),
                                 preferred_element_type=jnp.float32) \
        + bp_ref[...][None, :]
    qbar = jnp.mean(q4, axis=0, keepdims=True)
    second = jax.lax.dot_general(q4, q4, (((0,), (0,)), ((), ())),
                                 preferred_element_type=jnp.float32) / (B * Q)
    cov = second - jax.lax.dot_general(qbar, qbar, (((0,), (0,)), ((), ())),
                                       preferred_element_type=jnp.float32)
    wp = wp_ref[...]
    wc = jax.lax.dot_general(cov, wp, (((1,), (0,)), ((), ())),
                             preferred_element_type=jnp.float32)
    var = jnp.sum(wp * wc, axis=0)
    mu = jax.lax.dot_general(qbar, wp, (((1,), (0,)), ((), ())),
                             preferred_element_type=jnp.float32)[0] + bp_ref[...]
    inv = gamma_ref[...] / jnp.sqrt(var + 1e-5)
    pe = _leaky((pe_raw - mu[None, :]) * inv[None, :] + beta_ref[...][None, :])
    pe_ref[...] = pe.reshape(B, Q, 128)
    qplanar_ref[...] = jnp.transpose(q4.reshape(B, Q, 4), (0, 2, 1))


def _voxel_patches(voxel):
    v = voxel[..., 0]
    vp = jnp.pad(v, ((0, 0), (0, 2), (0, 2), (0, 2)))
    cols = []
    for i in range(3):
        for j in range(3):
            for k in range(3):
                c = lax.slice(vp, (0, i, j, k), (B, i + 32, j + 32, k + 32),
                              (1, 2, 2, 2))
                cols.append(c.reshape(B, 1, 4096))
    cols.append(jnp.zeros((B, 5, 4096), jnp.float32))
    return jnp.concatenate(cols, axis=1)


def _run_k3(local_pc, voxel, query, Wl1p, bl1, Wl2, bl2, Wvox2d, bvox,
            Wpp, bp, gamma, beta):
    voxel = _voxel_patches(voxel)
    return pl.pallas_call(
        _k3_body,
        out_shape=[
            jax.ShapeDtypeStruct((B, 1, 256), jnp.float32),
            jax.ShapeDtypeStruct((B, 1, 64), jnp.float32),
            jax.ShapeDtypeStruct((B, Q, 128), jnp.float32),
            jax.ShapeDtypeStruct((B, 4, Q), jnp.float32),
        ],
    )(local_pc, voxel, query, Wl1p, bl1, Wl2, bl2, Wvox2d, bvox,
      Wpp, bp, gamma, beta)


def _k5_body(kf_ref, pos_ref, pe_ref, gc_ref, lc_ref, vf_ref,
             wpos_ref, bpos_ref, wq_ref, wk_ref, wv_ref,
             wgc_ref, wlc_ref, wqf_ref, wpe_ref, wvf_ref, bd1_ref,
             wd2_ref, bd2_ref, out_ref):
    kf = kf_ref[...].reshape(256 * K, 128)
    pos = pos_ref[...].reshape(256 * K, 4)
    pos_enc = _leaky(jax.lax.dot_general(pos, wpos_ref[...],
                                         (((1,), (0,)), ((), ())),
                                         preferred_element_type=jnp.float32)
                     + bpos_ref[...][None, :])
    fq = jax.lax.dot_general(kf, wq_ref[...], (((1,), (0,)), ((), ())),
                             preferred_element_type=jnp.float32)
    fk = jax.lax.dot_general(pos_enc, wk_ref[...], (((1,), (0,)), ((), ())),
                             preferred_element_type=jnp.float32)
    fv = jax.lax.dot_general(kf + pos_enc, wv_ref[...], (((1,), (0,)), ((), ())),
                             preferred_element_type=jnp.float32)
    logits = jnp.sum(fq * fk, axis=1).reshape(256, K) / math.sqrt(float(C))
    mx = jnp.max(logits, axis=1, keepdims=True)
    e = jnp.exp(logits - mx)
    attn = e / jnp.sum(e, axis=1, keepdims=True)
    qf = jnp.sum(attn[:, :, None] * fv.reshape(256, K, 128), axis=1)

    cbase = (jax.lax.dot_general(gc_ref[0], wgc_ref[...],
                                 (((1,), (0,)), ((), ())),
                                 preferred_element_type=jnp.float32)
             + jax.lax.dot_general(lc_ref[0], wlc_ref[...],
                                   (((1,), (0,)), ((), ())),
                                   preferred_element_type=jnp.float32)
             + jax.lax.dot_general(vf_ref[0], wvf_ref[...],
                                   (((1,), (0,)), ((), ())),
                                   preferred_element_type=jnp.float32))
    pe = pe_ref[0]
    h = _leaky(jax.lax.dot_general(qf, wqf_ref[...], (((1,), (0,)), ((), ())),
                                   preferred_element_type=jnp.float32)
               + jax.lax.dot_general(pe, wpe_ref[...], (((1,), (0,)), ((), ())),
                                     preferred_element_type=jnp.float32)
               + cbase + bd1_ref[...][None, :])
    o = jax.lax.dot_general(h, wd2_ref[...], (((1,), (0,)), ((), ())),
                            preferred_element_type=jnp.float32) + bd2_ref[...]
    out_ref[0, 0] = o[:, 0]


def _run_k5(knn_feat, knn_pos, pe, gmax, lc, vf,
            WposT, bpos, WqT, WkT, WvT, Wgc, Wlc, Wqf, Wpe, Wvf, bd1,
            Wd2T, bd2):
    nt = Q // 256
    return pl.pallas_call(
        _k5_body,
        grid=(B, nt),
        in_specs=[
            pl.BlockSpec((256, K, 128), lambda b, t: (b * nt + t, 0, 0)),
            pl.BlockSpec((256, K, 4), lambda b, t: (b * nt + t, 0, 0)),
            pl.BlockSpec((1, 256, 128), lambda b, t: (b, t, 0)),
            pl.BlockSpec((1, 1, 256), lambda b, t: (b, 0, 0)),
            pl.BlockSpec((1, 1, 256), lambda b, t: (b, 0, 0)),
            pl.BlockSpec((1, 1, 64), lambda b, t: (b, 0, 0)),
            pl.BlockSpec((4, 128), lambda b, t: (0, 0)),
            pl.BlockSpec((128,), lambda b, t: (0,)),
            pl.BlockSpec((128, 128), lambda b, t: (0, 0)),
            pl.BlockSpec((128, 128), lambda b, t: (0, 0)),
            pl.BlockSpec((128, 128), lambda b, t: (0, 0)),
            pl.BlockSpec((256, 256), lambda b, t: (0, 0)),
            pl.BlockSpec((256, 256), lambda b, t: (0, 0)),
            pl.BlockSpec((128, 256), lambda b, t: (0, 0)),
            pl.BlockSpec((128, 256), lambda b, t: (0, 0)),
            pl.BlockSpec((64, 256), lambda b, t: (0, 0)),
            pl.BlockSpec((256,), lambda b, t: (0,)),
            pl.BlockSpec((256, 1), lambda b, t: (0, 0)),
            pl.BlockSpec((1,), lambda b, t: (0,)),
        ],
        out_specs=pl.BlockSpec((1, 1, 256), lambda b, t: (b, 0, t)),
        out_shape=jax.ShapeDtypeStruct((B, 1, Q), jnp.float32),
    )(knn_feat, knn_pos, pe, gmax, lc, vf,
      WposT, bpos, WqT, WkT, WvT, Wgc, Wlc, Wqf, Wpe, Wvf, bd1, Wd2T, bd2)


_THR_PAD = 2e-3
RPW = ROWS // 32


def _k4_body(gmin_hbm, qpl_hbm, planar_hbm, pf_hbm, kf_out, pos_out,
             xs, ys, zs, qbuf, gmin_row, gidbuf, dbuf, ibuf,
             idx16, featbuf, posbuf, sem):
    nc = 2
    wid = lax.axis_index("s") * nc + lax.axis_index("c")
    base = wid * RPW
    b = base // Q
    qoff = base % Q
    iota = lax.iota(jnp.int32, 16)
    inf = jnp.float32(jnp.inf)

    pltpu.sync_copy(planar_hbm.at[b, 0], xs)
    pltpu.sync_copy(planar_hbm.at[b, 1], ys)
    pltpu.sync_copy(planar_hbm.at[b, 2], zs)
    pltpu.sync_copy(qpl_hbm.at[b, :, pl.ds(qoff, RPW)], qbuf)
    i4 = iota * 4

    def zb(j, _):
        posbuf[pl.ds(pl.multiple_of(j * 16, 16), 16)] = jnp.zeros((16,),
                                                                  jnp.float32)
        return 0
    lax.fori_loop(0, 8, zb, 0)

    zero16 = jnp.zeros((16,), jnp.int32)

    def row_body(i, _):
        r = base + i
        pltpu.sync_copy(gmin_hbm.at[r], gmin_row)
        si = jnp.full((16,), i, jnp.int32)
        qx = plsc.load_gather(qbuf, [zero16, si])
        qy = plsc.load_gather(qbuf, [zero16 + 1, si])
        qz = plsc.load_gather(qbuf, [zero16 + 2, si])

        def pa(j, acc):
            return jnp.minimum(acc,
                               gmin_row[pl.ds(pl.multiple_of(j * 16, 16), 16)])
        acc = lax.fori_loop(0, NG // 16, pa, jnp.full((16,), inf, jnp.float32))
        thr = jnp.max(acc) + _THR_PAD

        def pb(j, cnt):
            v = gmin_row[pl.ds(pl.multiple_of(j * 16, 16), 16)]
            m = v <= thr
            mi = m.astype(jnp.int32)
            pos = cnt + plsc.cumsum(mi) - 1
            plsc.store_scatter(gidbuf, [pos], iota + j * 16, mask=m)
            return cnt + jnp.sum(mi)
        cnt = lax.fori_loop(0, NG // 16, pb, 0)

        def pc(ci, cnt2):
            gid = plsc.load_gather(gidbuf, [jnp.full((16,), ci, jnp.int32)])
            idxv = gid * 16 + iota
            px = plsc.load_gather(xs, [idxv])
            py = plsc.load_gather(ys, [idxv])
            pz = plsc.load_gather(zs, [idxv])
            dx, dy, dz = px - qx, py - qy, pz - qz
            d2 = dx * dx + dy * dy + dz * dz
            m = d2 <= thr
            mi = m.astype(jnp.int32)
            pos = cnt2 + plsc.cumsum(mi) - 1
            plsc.store_scatter(dbuf, [pos], d2, mask=m)
            plsc.store_scatter(ibuf, [pos], idxv, mask=m)
            return cnt2 + jnp.sum(mi)
        cnt2 = lax.fori_loop(0, cnt, pc, 0)

        def pd(ch, ti):
            tv, tidx = ti
            off = pl.multiple_of(ch * 16, 16)
            dv = dbuf[pl.ds(off, 16)]
            iv = ibuf[pl.ds(off, 16)]
            dv = jnp.where(iota < (cnt2 - off), dv, inf)
            sv, sx = plsc.sort_key_val(dv, iv)
            rv, ri = lax.rev(sv, (0,)), lax.rev(sx, (0,))
            keep = tv <= rv
            wv = jnp.where(keep, tv, rv)
            wi = jnp.where(keep, tidx, ri)
            sw = plsc.sort_key_val(wv, wi)
            return (sw[0], sw[1])
        tv0 = jnp.full((16,), inf, jnp.float32)
        ti0 = jnp.zeros((16,), jnp.int32)
        _, topi = lax.fori_loop(0, (cnt2 + 15) // 16, pd, (tv0, ti0))

        px = plsc.load_gather(xs, [topi])
        py = plsc.load_gather(ys, [topi])
        pz = plsc.load_gather(zs, [topi])
        plsc.store_scatter(posbuf, [i4], qx - px)
        plsc.store_scatter(posbuf, [i4 + 1], qy - py)
        plsc.store_scatter(posbuf, [i4 + 2], qz - pz)
        pltpu.sync_copy(posbuf, pos_out.at[r])
        idx16[...] = topi + b * N
        pltpu.async_copy(pf_hbm.at[idx16], featbuf, sem).wait()
        pltpu.sync_copy(featbuf, kf_out.at[r])
        return 0

    lax.fori_loop(0, RPW, row_body, 0)


def _run_k4(gmin, qplanar, planar, pf_flat):
    f32, i32 = jnp.float32, jnp.int32
    mesh = plsc.VectorSubcoreMesh(core_axis_name="c", subcore_axis_name="s")
    kfn = functools.partial(
        pl.kernel,
        mesh=mesh,
        compiler_params=pltpu.CompilerParams(needs_layout_passes=False),
        out_type=[
            jax.ShapeDtypeStruct((ROWS, K, 128), f32),
            jax.ShapeDtypeStruct((ROWS, 128), f32),
        ],
        scratch_types=[
            pltpu.VMEM((N,), f32), pltpu.VMEM((N,), f32), pltpu.VMEM((N,), f32),
            pltpu.VMEM((4, RPW), f32),
            pltpu.VMEM((NG,), f32),
            pltpu.VMEM((NG + 16,), i32),
            pltpu.VMEM((N + 16,), f32),
            pltpu.VMEM((N + 16,), i32),
            pltpu.VMEM((16,), i32),
            pltpu.VMEM((16, 128), f32),
            pltpu.VMEM((128,), f32),
            pltpu.SemaphoreType.DMA,
        ],
    )(_k4_body)
    kf, pos = kfn(gmin.reshape(ROWS, NG), qplanar, planar, pf_flat)
    return kf, pos[:, :K * 4].reshape(ROWS, K, 4)


def _knn_placeholder(gmin, query, global_pc, pf):
    d = (jnp.sum(query * query, axis=-1)[..., None]
         - 2.0 * jnp.einsum('bqd,bnd->bqn', query, global_pc)
         + jnp.sum(global_pc * global_pc, axis=-1)[:, None, :])
    _, idx = jax.lax.top_k(-d, K)
    knn_xyz = jax.vmap(lambda p, i: p[i])(global_pc, idx)
    knn_feat = jax.vmap(lambda f, i: f[i])(pf, idx)
    knn_pos = query[:, :, None, :] - knn_xyz
    knn_pos = jnp.concatenate(
        [knn_pos, jnp.zeros(knn_pos.shape[:-1] + (1,), jnp.float32)], axis=-1)
    return (knn_feat.reshape(ROWS, K, 128), knn_pos.reshape(ROWS, K, 4))


def kernel(global_pc, local_pc, query, voxel, Wg1, bg1, Wg2, bg2,
           Wl1, bl1, Wl2, bl2, Wvox, bvox, Wp, bp, gamma, beta,
           Wpos, bpos, Wq, Wk, Wv, Wd1, bd1, Wd2, bd2):
    f32 = jnp.float32
    pad1 = lambda w: jnp.concatenate([w, jnp.zeros((1,) + w.shape[1:], f32)], 0)
    Wg1p = pad1(Wg1)
    Wl1p = pad1(Wl1)
    Wvox2d = jnp.concatenate(
        [Wvox.reshape(64, 27).T, jnp.zeros((5, 64), f32)], axis=0)
    WpT = Wp.T
    Wpp = pad1(WpT)
    WposT = jnp.concatenate([Wpos.T, jnp.zeros((1, 128), f32)], 0)

    pf, gmax, planar = _run_k1(global_pc, Wg1p, bg1, Wg2, bg2)
    gmin = _run_k2(query, global_pc)
    lc, vf, pe, qplanar = _run_k3(local_pc, voxel, query, Wl1p, bl1, Wl2, bl2,
                                  Wvox2d, bvox, Wpp, bp, gamma, beta)

    knn_feat, knn_pos = _run_k4(gmin, qplanar, planar, pf.reshape(B * N, 128))

    out = _run_k5(knn_feat, knn_pos, pe, gmax, lc, vf,
                  WposT, bpos, Wq.T, Wk.T, Wv.T,
                  Wd1[:, 0:256].T, Wd1[:, 256:512].T, Wd1[:, 512:640].T,
                  Wd1[:, 640:768].T, Wd1[:, 768:832].T, bd1,
                  Wd2.T, bd2)
    return out


# speedup vs baseline: 10.2137x; 1.1929x over previous
import functools
import math

import jax
import jax.numpy as jnp
from jax import lax
from jax.experimental import pallas as pl
from jax.experimental.pallas import tpu as pltpu
from jax.experimental.pallas import tpu_sc as plsc

B, N, M, Q, D, K, C = 4, 8192, 2048, 2048, 32, 16, 128
NG = N // 16
ROWS = B * Q


def _leaky(x):
    return jnp.where(x >= 0, x, 0.2 * x)


def _k1_body(gpc_ref, wg1_ref, bg1_ref, wg2_ref, bg2_ref,
             pf_ref, gmax_ref, planar_ref):
    j = pl.program_id(1)
    g = gpc_ref[0]
    gp = jnp.concatenate([g, jnp.zeros((g.shape[0], 1), jnp.float32)], axis=1)
    pf = _leaky(jax.lax.dot_general(gp, wg1_ref[...],
                                    (((1,), (0,)), ((), ())),
                                    preferred_element_type=jnp.float32)
                + bg1_ref[...][None, :])
    pf_ref[0] = pf
    gg = _leaky(jax.lax.dot_general(pf, wg2_ref[...],
                                    (((1,), (0,)), ((), ())),
                                    preferred_element_type=jnp.float32)
                + bg2_ref[...][None, :])
    part = jnp.max(gg, axis=0, keepdims=True)

    @pl.when(j == 0)
    def _():
        gmax_ref[...] = jnp.full_like(gmax_ref, -jnp.inf)

    gmax_ref[0] = jnp.maximum(gmax_ref[0], part)
    planar_ref[0] = jnp.transpose(gp, (1, 0))


def _run_k1(global_pc, Wg1p, bg1, Wg2, bg2):
    return pl.pallas_call(
        _k1_body,
        grid=(B, N // 1024),
        in_specs=[
            pl.BlockSpec((1, 1024, 3), lambda b, j: (b, j, 0)),
            pl.BlockSpec((4, 128), lambda b, j: (0, 0)),
            pl.BlockSpec((128,), lambda b, j: (0,)),
            pl.BlockSpec((128, 256), lambda b, j: (0, 0)),
            pl.BlockSpec((256,), lambda b, j: (0,)),
        ],
        out_specs=[
            pl.BlockSpec((1, 1024, 128), lambda b, j: (b, j, 0)),
            pl.BlockSpec((1, 1, 256), lambda b, j: (b, 0, 0)),
            pl.BlockSpec((1, 4, 1024), lambda b, j: (b, 0, j)),
        ],
        out_shape=[
            jax.ShapeDtypeStruct((B, N, 128), jnp.float32),
            jax.ShapeDtypeStruct((B, 1, 256), jnp.float32),
            jax.ShapeDtypeStruct((B, 4, N), jnp.float32),
        ],
    )(global_pc, Wg1p, bg1, Wg2, bg2)


def _k2_body(q_ref, g_ref, gmin_ref):
    j = pl.program_id(2)
    q = q_ref[0]
    gt = g_ref[0]
    qp = jnp.concatenate([q, jnp.zeros((q.shape[0], 1), jnp.float32)], axis=1)
    gg = jnp.sum(gt * gt, axis=0, keepdims=True)
    cross = jax.lax.dot_general(qp, gt, (((1,), (0,)), ((), ())),
                                preferred_element_type=jnp.float32)
    part = gg - 2.0 * cross

    @pl.when(j == 0)
    def _():
        gmin_ref[0] = jnp.full_like(gmin_ref[0], jnp.inf)

    gmin_ref[0] = jnp.minimum(gmin_ref[0], part)


def _run_k2(query, planar):
    return pl.pallas_call(
        _k2_body,
        grid=(B, Q // 512, N // NG),
        in_specs=[
            pl.BlockSpec((1, 512, 3), lambda b, i, j: (b, i, 0)),
            pl.BlockSpec((1, 4, NG), lambda b, i, j: (b, 0, j)),
        ],
        out_specs=pl.BlockSpec((1, 512, NG), lambda b, i, j: (b, i, 0)),
        out_shape=jax.ShapeDtypeStruct((B, Q, NG), jnp.float32),
    )(query, planar)


def _k3_body(lpc_ref, vox_ref, q_ref,
             wl1_ref, bl1_ref, wl2_ref, bl2_ref,
             wvox_ref, bvox_ref, wp_ref, bp_ref, gamma_ref, beta_ref,
             lc_ref, vf_ref, pe_ref, qplanar_ref):
    lp = lpc_ref[...].reshape(B * M, 3)
    lp4 = jnp.concatenate([lp, jnp.zeros((B * M, 1), jnp.float32)], axis=1)
    lh = _leaky(jax.lax.dot_general(lp4, wl1_ref[...], (((1,), (0,)), ((), ())),
                                    preferred_element_type=jnp.float32)
                + bl1_ref[...][None, :])
    l2 = _leaky(jax.lax.dot_general(lh, wl2_ref[...], (((1,), (0,)), ((), ())),
                                    preferred_element_type=jnp.float32)
                + bl2_ref[...][None, :])
    lc_ref[...] = jnp.max(l2.reshape(B, M, 256), axis=1, keepdims=True)

    w2d = wvox_ref[...]
    for b in range(B):
        conv = jax.lax.dot_general(w2d, vox_ref[b], (((0,), (0,)), ((), ())),
                                   preferred_element_type=jnp.float32)
        conv = _leaky(conv + bvox_ref[...][:, None])
        vf_ref[b, 0] = jnp.mean(conv, axis=1)

    q = q_ref[...].reshape(B * Q, 3)
    q4 = jnp.concatenate([q, jnp.zeros((B * Q, 1), jnp.float32)], axis=1)
    pe_raw = jax.lax.dot_general(q4, wp_ref[...], (((1,), (0,)), ((), ())),
                                 preferred_element_type=jnp.float32) \
        + bp_ref[...][None, :]
    qbar = jnp.mean(q4, axis=0, keepdims=True)
    second = jax.lax.dot_general(q4, q4, (((0,), (0,)), ((), ())),
                                 preferred_element_type=jnp.float32) / (B * Q)
    cov = second - jax.lax.dot_general(qbar, qbar, (((0,), (0,)), ((), ())),
                                       preferred_element_type=jnp.float32)
    wp = wp_ref[...]
    wc = jax.lax.dot_general(cov, wp, (((1,), (0,)), ((), ())),
                             preferred_element_type=jnp.float32)
    var = jnp.sum(wp * wc, axis=0)
    mu = jax.lax.dot_general(qbar, wp, (((1,), (0,)), ((), ())),
                             preferred_element_type=jnp.float32)[0] + bp_ref[...]
    inv = gamma_ref[...] / jnp.sqrt(var + 1e-5)
    pe = _leaky((pe_raw - mu[None, :]) * inv[None, :] + beta_ref[...][None, :])
    pe_ref[...] = pe.reshape(B, Q, 128)
    qplanar_ref[...] = jnp.transpose(q4.reshape(B, Q, 4), (0, 2, 1))


def _voxel_patches(voxel):
    v = voxel[..., 0]
    vp = jnp.pad(v, ((0, 0), (0, 2), (0, 2), (0, 2)))
    cols = []
    for i in range(3):
        for j in range(3):
            for k in range(3):
                c = lax.slice(vp, (0, i, j, k), (B, i + 32, j + 32, k + 32),
                              (1, 2, 2, 2))
                cols.append(c.reshape(B, 1, 4096))
    cols.append(jnp.zeros((B, 5, 4096), jnp.float32))
    return jnp.concatenate(cols, axis=1)


def _run_k3(local_pc, voxel, query, Wl1p, bl1, Wl2, bl2, Wvox2d, bvox,
            Wpp, bp, gamma, beta):
    voxel = _voxel_patches(voxel)
    return pl.pallas_call(
        _k3_body,
        out_shape=[
            jax.ShapeDtypeStruct((B, 1, 256), jnp.float32),
            jax.ShapeDtypeStruct((B, 1, 64), jnp.float32),
            jax.ShapeDtypeStruct((B, Q, 128), jnp.float32),
            jax.ShapeDtypeStruct((B, 4, Q), jnp.float32),
        ],
    )(local_pc, voxel, query, Wl1p, bl1, Wl2, bl2, Wvox2d, bvox,
      Wpp, bp, gamma, beta)


def _k5_body(kf_ref, pos_ref, pe_ref, gc_ref, lc_ref, vf_ref,
             wpos_ref, bpos_ref, wq_ref, wk_ref, wv_ref,
             wgc_ref, wlc_ref, wqf_ref, wpe_ref, wvf_ref, bd1_ref,
             wd2_ref, bd2_ref, out_ref):
    kf = kf_ref[...].reshape(256 * K, 128)
    pos = pos_ref[...].reshape(256 * K, 4)
    pos_enc = _leaky(jax.lax.dot_general(pos, wpos_ref[...],
                                         (((1,), (0,)), ((), ())),
                                         preferred_element_type=jnp.float32)
                     + bpos_ref[...][None, :])
    fq = jax.lax.dot_general(kf, wq_ref[...], (((1,), (0,)), ((), ())),
                             preferred_element_type=jnp.float32)
    fk = jax.lax.dot_general(pos_enc, wk_ref[...], (((1,), (0,)), ((), ())),
                             preferred_element_type=jnp.float32)
    fv = jax.lax.dot_general(kf + pos_enc, wv_ref[...], (((1,), (0,)), ((), ())),
                             preferred_element_type=jnp.float32)
    logits = jnp.sum(fq * fk, axis=1).reshape(256, K) / math.sqrt(float(C))
    mx = jnp.max(logits, axis=1, keepdims=True)
    e = jnp.exp(logits - mx)
    attn = e / jnp.sum(e, axis=1, keepdims=True)
    qf = jnp.sum(attn[:, :, None] * fv.reshape(256, K, 128), axis=1)

    cbase = (jax.lax.dot_general(gc_ref[0], wgc_ref[...],
                                 (((1,), (0,)), ((), ())),
                                 preferred_element_type=jnp.float32)
             + jax.lax.dot_general(lc_ref[0], wlc_ref[...],
                                   (((1,), (0,)), ((), ())),
                                   preferred_element_type=jnp.float32)
             + jax.lax.dot_general(vf_ref[0], wvf_ref[...],
                                   (((1,), (0,)), ((), ())),
                                   preferred_element_type=jnp.float32))
    pe = pe_ref[0]
    h = _leaky(jax.lax.dot_general(qf, wqf_ref[...], (((1,), (0,)), ((), ())),
                                   preferred_element_type=jnp.float32)
               + jax.lax.dot_general(pe, wpe_ref[...], (((1,), (0,)), ((), ())),
                                     preferred_element_type=jnp.float32)
               + cbase + bd1_ref[...][None, :])
    o = jax.lax.dot_general(h, wd2_ref[...], (((1,), (0,)), ((), ())),
                            preferred_element_type=jnp.float32) + bd2_ref[...]
    out_ref[0, 0] = o[:, 0]


def _run_k5(knn_feat, knn_pos, pe, gmax, lc, vf,
            WposT, bpos, WqT, WkT, WvT, Wgc, Wlc, Wqf, Wpe, Wvf, bd1,
            Wd2T, bd2):
    nt = Q // 256
    return pl.pallas_call(
        _k5_body,
        grid=(B, nt),
        in_specs=[
            pl.BlockSpec((256, K, 128), lambda b, t: (b * nt + t, 0, 0)),
            pl.BlockSpec((256, K, 4), lambda b, t: (b * nt + t, 0, 0)),
            pl.BlockSpec((1, 256, 128), lambda b, t: (b, t, 0)),
            pl.BlockSpec((1, 1, 256), lambda b, t: (b, 0, 0)),
            pl.BlockSpec((1, 1, 256), lambda b, t: (b, 0, 0)),
            pl.BlockSpec((1, 1, 64), lambda b, t: (b, 0, 0)),
            pl.BlockSpec((4, 128), lambda b, t: (0, 0)),
            pl.BlockSpec((128,), lambda b, t: (0,)),
            pl.BlockSpec((128, 128), lambda b, t: (0, 0)),
            pl.BlockSpec((128, 128), lambda b, t: (0, 0)),
            pl.BlockSpec((128, 128), lambda b, t: (0, 0)),
            pl.BlockSpec((256, 256), lambda b, t: (0, 0)),
            pl.BlockSpec((256, 256), lambda b, t: (0, 0)),
            pl.BlockSpec((128, 256), lambda b, t: (0, 0)),
            pl.BlockSpec((128, 256), lambda b, t: (0, 0)),
            pl.BlockSpec((64, 256), lambda b, t: (0, 0)),
            pl.BlockSpec((256,), lambda b, t: (0,)),
            pl.BlockSpec((256, 1), lambda b, t: (0, 0)),
            pl.BlockSpec((1,), lambda b, t: (0,)),
        ],
        out_specs=pl.BlockSpec((1, 1, 256), lambda b, t: (b, 0, t)),
        out_shape=jax.ShapeDtypeStruct((B, 1, Q), jnp.float32),
    )(knn_feat, knn_pos, pe, gmax, lc, vf,
      WposT, bpos, WqT, WkT, WvT, Wgc, Wlc, Wqf, Wpe, Wvf, bd1, Wd2T, bd2)


_THR_PAD = 2e-3
RPW = ROWS // 32
BR = 8
NB = RPW // BR


def _lane(v, j):
    return lax.squeeze(lax.slice(v, (j,), (j + 1,)), (0,))


def _k4_body(gmin_hbm, qpl_hbm, planar_hbm, pf_hbm, kf_out, pos_out,
             xs, ys, zs, qbuf, gm, gidbuf, dbuf, ibuf,
             idxb0, idxb1, fb0, fb1, pb0, pb1,
             semm0, semm1, semg0, semg1, semw0, semw1, semp0, semp1):
    nc = 2
    wid = lax.axis_index("s") * nc + lax.axis_index("c")
    base = wid * RPW
    b = base // Q
    qoff = base % Q
    iota = lax.iota(jnp.int32, 16)
    inf = jnp.float32(jnp.inf)
    zero16 = jnp.zeros((16,), jnp.int32)
    i4 = iota * 4

    pltpu.sync_copy(planar_hbm.at[b, 0], xs)
    pltpu.sync_copy(planar_hbm.at[b, 1], ys)
    pltpu.sync_copy(planar_hbm.at[b, 2], zs)
    pltpu.sync_copy(qpl_hbm.at[b, :, pl.ds(qoff, RPW)], qbuf)

    def zb(j, _):
        o = pl.ds(pl.multiple_of(j * 16, 16), 16)
        pb0[o] = jnp.zeros((16,), jnp.float32)
        pb1[o] = jnp.zeros((16,), jnp.float32)
        return 0
    lax.fori_loop(0, BR * 8, zb, 0)

    def parity(p, fn0, fn1):
        @pl.when(p == 0)
        def _():
            fn0()

        @pl.when(p == 1)
        def _():
            fn1()

    pltpu.async_copy(gmin_hbm.at[pl.ds(base, BR)], gm.at[pl.ds(0, BR)], semm0)

    def block_body(k, _):
        p = k % 2
        r0 = base + k * BR

        parity(p,
               lambda: pltpu.make_async_copy(gmin_hbm.at[pl.ds(r0, BR)],
                                             gm.at[pl.ds(0, BR)],
                                             semm0).wait(),
               lambda: pltpu.make_async_copy(gmin_hbm.at[pl.ds(r0, BR)],
                                             gm.at[pl.ds(BR, BR)],
                                             semm1).wait())

        @pl.when(k + 1 < NB)
        def _():
            rn = r0 + BR
            parity(1 - p,
                   lambda: pltpu.async_copy(gmin_hbm.at[pl.ds(rn, BR)],
                                            gm.at[pl.ds(0, BR)], semm0),
                   lambda: pltpu.async_copy(gmin_hbm.at[pl.ds(rn, BR)],
                                            gm.at[pl.ds(BR, BR)], semm1))

        @pl.when(k >= 2)
        def _():
            parity(p,
                   lambda: (pltpu.make_async_copy(
                       fb0, kf_out.at[pl.ds((r0 - 2 * BR) * K, BR * K)],
                       semw0).wait(),
                       pltpu.make_async_copy(
                       pb0, pos_out.at[pl.ds((r0 - 2 * BR) * 128, BR * 128)],
                       semp0).wait()),
                   lambda: (pltpu.make_async_copy(
                       fb1, kf_out.at[pl.ds((r0 - 2 * BR) * K, BR * K)],
                       semw1).wait(),
                       pltpu.make_async_copy(
                       pb1, pos_out.at[pl.ds((r0 - 2 * BR) * 128, BR * 128)],
                       semp1).wait()))

        def row_body(rb, _):
            i = k * BR + rb
            ri = p * BR + rb
            si = jnp.full((16,), i, jnp.int32)
            qx = plsc.load_gather(qbuf, [zero16, si])
            qy = plsc.load_gather(qbuf, [zero16 + 1, si])
            qz = plsc.load_gather(qbuf, [zero16 + 2, si])

            def gmrow(j):
                return gm[ri, pl.ds(pl.multiple_of(j * 16, 16), 16)]

            def pa(j, acc):
                return jnp.minimum(acc, gmrow(j))
            acc = lax.fori_loop(0, NG // 16, pa,
                                jnp.full((16,), inf, jnp.float32))
            thr = jnp.max(acc) + _THR_PAD
            thrv = thr + (qx * qx + qy * qy + qz * qz)

            def pb_(j, cnt):
                v = gmrow(j)
                m = v <= thr
                pos = cnt + plsc.cumsum(m.astype(jnp.int32)) - 1
                plsc.store_scatter(gidbuf, [pos], iota + j * 16, mask=m)
                return _lane(pos, 15) + 1
            cnt = lax.fori_loop(0, NG // 16, pb_, 0)

            def pc(ci, _c):
                gid = plsc.load_gather(gidbuf,
                                       [jnp.full((16,), ci, jnp.int32)])
                idxv = gid + iota * NG
                px = plsc.load_gather(xs, [idxv])
                py = plsc.load_gather(ys, [idxv])
                pz = plsc.load_gather(zs, [idxv])
                dx, dy, dz = px - qx, py - qy, pz - qz
                d2 = dx * dx + dy * dy + dz * dz
                o = pl.ds(pl.multiple_of(ci * 16, 16), 16)
                dbuf[o] = jnp.where(d2 <= thrv, d2, inf)
                ibuf[o] = idxv
                return 0
            lax.fori_loop(0, cnt, pc, 0)

            def pd(ch, ti):
                tv, tidx = ti
                off = pl.ds(pl.multiple_of(ch * 16, 16), 16)
                dv = dbuf[off]
                hit = plsc.all_reduce_population_count(dv < _lane(tv, 15))

                def merge():
                    iv = ibuf[off]
                    sv, sx = plsc.sort_key_val(dv, iv)
                    rv, ri = lax.rev(sv, (0,)), lax.rev(sx, (0,))
                    keep = tv <= rv
                    sw = plsc.sort_key_val(jnp.where(keep, tv, rv),
                                           jnp.where(keep, tidx, ri))
                    return (sw[0], sw[1])
                return lax.cond(_lane(hit, 0) > 0, merge, lambda: (tv, tidx))
            tv0 = jnp.full((16,), inf, jnp.float32)
            _, topi = lax.fori_loop(0, cnt, pd, (tv0, zero16))

            px = plsc.load_gather(xs, [topi])
            py = plsc.load_gather(ys, [topi])
            pz = plsc.load_gather(zs, [topi])
            pidx = i4 + rb * 128
            io = pl.ds(pl.multiple_of(rb * 16, 16), 16)
            gi = topi + b * N
            parity(p,
                   lambda: (plsc.store_scatter(pb0, [pidx], qx - px),
                            plsc.store_scatter(pb0, [pidx + 1], qy - py),
                            plsc.store_scatter(pb0, [pidx + 2], qz - pz),
                            idxb0.__setitem__(io, gi)),
                   lambda: (plsc.store_scatter(pb1, [pidx], qx - px),
                            plsc.store_scatter(pb1, [pidx + 1], qy - py),
                            plsc.store_scatter(pb1, [pidx + 2], qz - pz),
                            idxb1.__setitem__(io, gi)))
            return 0

        lax.fori_loop(0, BR, row_body, 0)

        parity(p,
               lambda: (pltpu.async_copy(pf_hbm.at[idxb0], fb0, semg0),
                        pltpu.async_copy(
                            pb0, pos_out.at[pl.ds(r0 * 128, BR * 128)],
                            semp0)),
               lambda: (pltpu.async_copy(pf_hbm.at[idxb1], fb1, semg1),
                        pltpu.async_copy(
                            pb1, pos_out.at[pl.ds(r0 * 128, BR * 128)],
                            semp1)))

        @pl.when(k >= 1)
        def _():
            rp = r0 - BR
            parity(1 - p,
                   lambda: (pltpu.make_async_copy(pf_hbm.at[idxb0], fb0,
                                                  semg0).wait(),
                            pltpu.async_copy(
                                fb0, kf_out.at[pl.ds(rp * K, BR * K)], semw0)),
                   lambda: (pltpu.make_async_copy(pf_hbm.at[idxb1], fb1,
                                                  semg1).wait(),
                            pltpu.async_copy(
                                fb1, kf_out.at[pl.ds(rp * K, BR * K)], semw1)))
        return 0

    lax.fori_loop(0, NB, block_body, 0)

    rl = base + (NB - 1) * BR
    pltpu.make_async_copy(pf_hbm.at[idxb1], fb1, semg1).wait()
    pltpu.sync_copy(fb1, kf_out.at[pl.ds(rl * K, BR * K)])
    pltpu.make_async_copy(fb0, kf_out.at[pl.ds((rl - BR) * K, BR * K)],
                          semw0).wait()
    pltpu.make_async_copy(pb0, pos_out.at[pl.ds((rl - BR) * 128, BR * 128)],
                          semp0).wait()
    pltpu.make_async_copy(pb1, pos_out.at[pl.ds(rl * 128, BR * 128)],
                          semp1).wait()


def _run_k4(gmin, qplanar, planar, pf_flat):
    f32, i32 = jnp.float32, jnp.int32
    mesh = plsc.VectorSubcoreMesh(core_axis_name="c", subcore_axis_name="s")
    kfn = functools.partial(
        pl.kernel,
        mesh=mesh,
        compiler_params=pltpu.CompilerParams(needs_layout_passes=False),
        out_type=[
            jax.ShapeDtypeStruct((ROWS * K, 128), f32),
            jax.ShapeDtypeStruct((ROWS * 128,), f32),
        ],
        scratch_types=[
            pltpu.VMEM((N,), f32), pltpu.VMEM((N,), f32), pltpu.VMEM((N,), f32),
            pltpu.VMEM((4, RPW), f32),
            pltpu.VMEM((2 * BR, NG), f32),
            pltpu.VMEM((NG + 16,), i32),
            pltpu.VMEM((N + 16,), f32),
            pltpu.VMEM((N + 16,), i32),
            pltpu.VMEM((BR * K,), i32), pltpu.VMEM((BR * K,), i32),
            pltpu.VMEM((BR * K, 128), f32), pltpu.VMEM((BR * K, 128), f32),
            pltpu.VMEM((BR * 128,), f32), pltpu.VMEM((BR * 128,), f32),
            pltpu.SemaphoreType.DMA, pltpu.SemaphoreType.DMA,
            pltpu.SemaphoreType.DMA, pltpu.SemaphoreType.DMA,
            pltpu.SemaphoreType.DMA, pltpu.SemaphoreType.DMA,
            pltpu.SemaphoreType.DMA, pltpu.SemaphoreType.DMA,
        ],
    )(_k4_body)
    kf, pos = kfn(gmin.reshape(ROWS, NG), qplanar, planar, pf_flat)
    return (kf.reshape(ROWS, K, 128),
            pos.reshape(ROWS, 128)[:, :K * 4].reshape(ROWS, K, 4))


def _knn_placeholder(gmin, query, global_pc, pf):
    d = (jnp.sum(query * query, axis=-1)[..., None]
         - 2.0 * jnp.einsum('bqd,bnd->bqn', query, global_pc)
         + jnp.sum(global_pc * global_pc, axis=-1)[:, None, :])
    _, idx = jax.lax.top_k(-d, K)
    knn_xyz = jax.vmap(lambda p, i: p[i])(global_pc, idx)
    knn_feat = jax.vmap(lambda f, i: f[i])(pf, idx)
    knn_pos = query[:, :, None, :] - knn_xyz
    knn_pos = jnp.concatenate(
        [knn_pos, jnp.zeros(knn_pos.shape[:-1] + (1,), jnp.float32)], axis=-1)
    return (knn_feat.reshape(ROWS, K, 128), knn_pos.reshape(ROWS, K, 4))


def kernel(global_pc, local_pc, query, voxel, Wg1, bg1, Wg2, bg2,
           Wl1, bl1, Wl2, bl2, Wvox, bvox, Wp, bp, gamma, beta,
           Wpos, bpos, Wq, Wk, Wv, Wd1, bd1, Wd2, bd2):
    f32 = jnp.float32
    pad1 = lambda w: jnp.concatenate([w, jnp.zeros((1,) + w.shape[1:], f32)], 0)
    Wg1p = pad1(Wg1)
    Wl1p = pad1(Wl1)
    Wvox2d = jnp.concatenate(
        [Wvox.reshape(64, 27).T, jnp.zeros((5, 64), f32)], axis=0)
    WpT = Wp.T
    Wpp = pad1(WpT)
    WposT = jnp.concatenate([Wpos.T, jnp.zeros((1, 128), f32)], 0)

    pf, gmax, planar = _run_k1(global_pc, Wg1p, bg1, Wg2, bg2)
    gmin = _run_k2(query, planar)
    lc, vf, pe, qplanar = _run_k3(local_pc, voxel, query, Wl1p, bl1, Wl2, bl2,
                                  Wvox2d, bvox, Wpp, bp, gamma, beta)

    knn_feat, knn_pos = _run_k4(gmin, qplanar, planar, pf.reshape(B * N, 128))

    out = _run_k5(knn_feat, knn_pos, pe, gmax, lc, vf,
                  WposT, bpos, Wq.T, Wk.T, Wv.T,
                  Wd1[:, 0:256].T, Wd1[:, 256:512].T, Wd1[:, 512:640].T,
                  Wd1[:, 640:768].T, Wd1[:, 768:832].T, bd1,
                  Wd2.T, bd2)
    return out


# speedup vs baseline: 12.1394x; 1.1885x over previous
import functools
import math

import jax
import jax.numpy as jnp
from jax import lax
from jax.experimental import pallas as pl
from jax.experimental.pallas import tpu as pltpu
from jax.experimental.pallas import tpu_sc as plsc

B, N, M, Q, D, K, C = 4, 8192, 2048, 2048, 32, 16, 128
NG = N // 16
ROWS = B * Q


def _leaky(x):
    return jnp.where(x >= 0, x, 0.2 * x)


def _k1_body(gpc_ref, wg1_ref, bg1_ref, wg2_ref, bg2_ref,
             pf_ref, gmax_ref, planar_ref):
    j = pl.program_id(1)
    g = gpc_ref[0]
    gp = jnp.concatenate([g, jnp.zeros((g.shape[0], 1), jnp.float32)], axis=1)
    pf = _leaky(jax.lax.dot_general(gp, wg1_ref[...],
                                    (((1,), (0,)), ((), ())),
                                    preferred_element_type=jnp.float32)
                + bg1_ref[...][None, :])
    pf_ref[0] = pf
    gg = _leaky(jax.lax.dot_general(pf, wg2_ref[...],
                                    (((1,), (0,)), ((), ())),
                                    preferred_element_type=jnp.float32)
                + bg2_ref[...][None, :])
    part = jnp.max(gg, axis=0, keepdims=True)

    @pl.when(j == 0)
    def _():
        gmax_ref[...] = jnp.full_like(gmax_ref, -jnp.inf)

    gmax_ref[0] = jnp.maximum(gmax_ref[0], part)
    planar_ref[0] = jnp.transpose(gp, (1, 0))


def _run_k1(global_pc, Wg1p, bg1, Wg2, bg2):
    return pl.pallas_call(
        _k1_body,
        grid=(B, N // 1024),
        in_specs=[
            pl.BlockSpec((1, 1024, 3), lambda b, j: (b, j, 0)),
            pl.BlockSpec((4, 128), lambda b, j: (0, 0)),
            pl.BlockSpec((128,), lambda b, j: (0,)),
            pl.BlockSpec((128, 256), lambda b, j: (0, 0)),
            pl.BlockSpec((256,), lambda b, j: (0,)),
        ],
        out_specs=[
            pl.BlockSpec((1, 1024, 128), lambda b, j: (b, j, 0)),
            pl.BlockSpec((1, 1, 256), lambda b, j: (b, 0, 0)),
            pl.BlockSpec((1, 4, 1024), lambda b, j: (b, 0, j)),
        ],
        out_shape=[
            jax.ShapeDtypeStruct((B, N, 128), jnp.float32),
            jax.ShapeDtypeStruct((B, 1, 256), jnp.float32),
            jax.ShapeDtypeStruct((B, 4, N), jnp.float32),
        ],
    )(global_pc, Wg1p, bg1, Wg2, bg2)


def _k2_body(q_ref, g_ref, gmin_ref):
    j = pl.program_id(2)
    q = q_ref[0]
    gt = g_ref[0]
    qp = jnp.concatenate([q, jnp.zeros((q.shape[0], 1), jnp.float32)], axis=1)
    gg = jnp.sum(gt * gt, axis=0, keepdims=True)
    cross = jax.lax.dot_general(qp, gt, (((1,), (0,)), ((), ())),
                                preferred_element_type=jnp.float32)
    part = gg - 2.0 * cross

    @pl.when(j == 0)
    def _():
        gmin_ref[0] = jnp.full_like(gmin_ref[0], jnp.inf)

    gmin_ref[0] = jnp.minimum(gmin_ref[0], part)


def _run_k2(query, planar):
    return pl.pallas_call(
        _k2_body,
        grid=(B, Q // 512, N // NG),
        in_specs=[
            pl.BlockSpec((1, 512, 3), lambda b, i, j: (b, i, 0)),
            pl.BlockSpec((1, 4, NG), lambda b, i, j: (b, 0, j)),
        ],
        out_specs=pl.BlockSpec((1, 512, NG), lambda b, i, j: (b, i, 0)),
        out_shape=jax.ShapeDtypeStruct((B, Q, NG), jnp.float32),
    )(query, planar)


def _k3_body(lpc_ref, vox_ref, q_ref,
             wl1_ref, bl1_ref, wl2_ref, bl2_ref,
             wvox_ref, bvox_ref, wp_ref, bp_ref, gamma_ref, beta_ref,
             lc_ref, vf_ref, pe_ref, qplanar_ref):
    lp = lpc_ref[...].reshape(B * M, 3)
    lp4 = jnp.concatenate([lp, jnp.zeros((B * M, 1), jnp.float32)], axis=1)
    lh = _leaky(jax.lax.dot_general(lp4, wl1_ref[...], (((1,), (0,)), ((), ())),
                                    preferred_element_type=jnp.float32)
                + bl1_ref[...][None, :])
    l2 = _leaky(jax.lax.dot_general(lh, wl2_ref[...], (((1,), (0,)), ((), ())),
                                    preferred_element_type=jnp.float32)
                + bl2_ref[...][None, :])
    lc_ref[...] = jnp.max(l2.reshape(B, M, 256), axis=1, keepdims=True)

    w2d = wvox_ref[...]
    for b in range(B):
        conv = jax.lax.dot_general(w2d, vox_ref[b], (((0,), (0,)), ((), ())),
                                   preferred_element_type=jnp.float32)
        conv = _leaky(conv + bvox_ref[...][:, None])
        vf_ref[b, 0] = jnp.mean(conv, axis=1)

    q = q_ref[...].reshape(B * Q, 3)
    q4 = jnp.concatenate([q, jnp.zeros((B * Q, 1), jnp.float32)], axis=1)
    pe_raw = jax.lax.dot_general(q4, wp_ref[...], (((1,), (0,)), ((), ())),
                                 preferred_element_type=jnp.float32) \
        + bp_ref[...][None, :]
    qbar = jnp.mean(q4, axis=0, keepdims=True)
    second = jax.lax.dot_general(q4, q4, (((0,), (0,)), ((), ())),
                                 preferred_element_type=jnp.float32) / (B * Q)
    cov = second - jax.lax.dot_general(qbar, qbar, (((0,), (0,)), ((), ())),
                                       preferred_element_type=jnp.float32)
    wp = wp_ref[...]
    wc = jax.lax.dot_general(cov, wp, (((1,), (0,)), ((), ())),
                             preferred_element_type=jnp.float32)
    var = jnp.sum(wp * wc, axis=0)
    mu = jax.lax.dot_general(qbar, wp, (((1,), (0,)), ((), ())),
                             preferred_element_type=jnp.float32)[0] + bp_ref[...]
    inv = gamma_ref[...] / jnp.sqrt(var + 1e-5)
    pe = _leaky((pe_raw - mu[None, :]) * inv[None, :] + beta_ref[...][None, :])
    pe_ref[...] = pe.reshape(B, Q, 128)
    qplanar_ref[...] = jnp.transpose(q4.reshape(B, Q, 4), (0, 2, 1))


def _voxel_patches(voxel):
    v = voxel[..., 0]
    vp = jnp.pad(v, ((0, 0), (0, 2), (0, 2), (0, 2)))
    cols = []
    for i in range(3):
        for j in range(3):
            for k in range(3):
                c = lax.slice(vp, (0, i, j, k), (B, i + 32, j + 32, k + 32),
                              (1, 2, 2, 2))
                cols.append(c.reshape(B, 1, 4096))
    cols.append(jnp.zeros((B, 5, 4096), jnp.float32))
    return jnp.concatenate(cols, axis=1)


def _run_k3(local_pc, voxel, query, Wl1p, bl1, Wl2, bl2, Wvox2d, bvox,
            Wpp, bp, gamma, beta):
    voxel = _voxel_patches(voxel)
    return pl.pallas_call(
        _k3_body,
        out_shape=[
            jax.ShapeDtypeStruct((B, 1, 256), jnp.float32),
            jax.ShapeDtypeStruct((B, 1, 64), jnp.float32),
            jax.ShapeDtypeStruct((B, Q, 128), jnp.float32),
            jax.ShapeDtypeStruct((B, 4, Q), jnp.float32),
        ],
    )(local_pc, voxel, query, Wl1p, bl1, Wl2, bl2, Wvox2d, bvox,
      Wpp, bp, gamma, beta)


def _k5_body(kf_ref, pos_ref, pe_ref, gc_ref, lc_ref, vf_ref,
             wpos_ref, bpos_ref, wq_ref, wk_ref, wv_ref,
             wgc_ref, wlc_ref, wqf_ref, wpe_ref, wvf_ref, bd1_ref,
             wd2_ref, bd2_ref, out_ref):
    kf = kf_ref[...].reshape(256 * K, 128)
    pos = pos_ref[...].reshape(256 * K, 4)
    pos_enc = _leaky(jax.lax.dot_general(pos, wpos_ref[...],
                                         (((1,), (0,)), ((), ())),
                                         preferred_element_type=jnp.float32)
                     + bpos_ref[...][None, :])
    fq = jax.lax.dot_general(kf, wq_ref[...], (((1,), (0,)), ((), ())),
                             preferred_element_type=jnp.float32)
    fk = jax.lax.dot_general(pos_enc, wk_ref[...], (((1,), (0,)), ((), ())),
                             preferred_element_type=jnp.float32)
    fv = jax.lax.dot_general(kf + pos_enc, wv_ref[...], (((1,), (0,)), ((), ())),
                             preferred_element_type=jnp.float32)
    logits = jnp.sum(fq * fk, axis=1).reshape(256, K) / math.sqrt(float(C))
    mx = jnp.max(logits, axis=1, keepdims=True)
    e = jnp.exp(logits - mx)
    attn = e / jnp.sum(e, axis=1, keepdims=True)
    qf = jnp.sum(attn[:, :, None] * fv.reshape(256, K, 128), axis=1)

    cbase = (jax.lax.dot_general(gc_ref[0], wgc_ref[...],
                                 (((1,), (0,)), ((), ())),
                                 preferred_element_type=jnp.float32)
             + jax.lax.dot_general(lc_ref[0], wlc_ref[...],
                                   (((1,), (0,)), ((), ())),
                                   preferred_element_type=jnp.float32)
             + jax.lax.dot_general(vf_ref[0], wvf_ref[...],
                                   (((1,), (0,)), ((), ())),
                                   preferred_element_type=jnp.float32))
    pe = pe_ref[0]
    h = _leaky(jax.lax.dot_general(qf, wqf_ref[...], (((1,), (0,)), ((), ())),
                                   preferred_element_type=jnp.float32)
               + jax.lax.dot_general(pe, wpe_ref[...], (((1,), (0,)), ((), ())),
                                     preferred_element_type=jnp.float32)
               + cbase + bd1_ref[...][None, :])
    o = jax.lax.dot_general(h, wd2_ref[...], (((1,), (0,)), ((), ())),
                            preferred_element_type=jnp.float32) + bd2_ref[...]
    out_ref[0, 0] = o[:, 0]


def _run_k5(knn_feat, knn_pos, pe, gmax, lc, vf,
            WposT, bpos, WqT, WkT, WvT, Wgc, Wlc, Wqf, Wpe, Wvf, bd1,
            Wd2T, bd2):
    nt = Q // 256
    return pl.pallas_call(
        _k5_body,
        grid=(B, nt),
        in_specs=[
            pl.BlockSpec((256, K, 128), lambda b, t: (b * nt + t, 0, 0)),
            pl.BlockSpec((256, K, 4), lambda b, t: (b * nt + t, 0, 0)),
            pl.BlockSpec((1, 256, 128), lambda b, t: (b, t, 0)),
            pl.BlockSpec((1, 1, 256), lambda b, t: (b, 0, 0)),
            pl.BlockSpec((1, 1, 256), lambda b, t: (b, 0, 0)),
            pl.BlockSpec((1, 1, 64), lambda b, t: (b, 0, 0)),
            pl.BlockSpec((4, 128), lambda b, t: (0, 0)),
            pl.BlockSpec((128,), lambda b, t: (0,)),
            pl.BlockSpec((128, 128), lambda b, t: (0, 0)),
            pl.BlockSpec((128, 128), lambda b, t: (0, 0)),
            pl.BlockSpec((128, 128), lambda b, t: (0, 0)),
            pl.BlockSpec((256, 256), lambda b, t: (0, 0)),
            pl.BlockSpec((256, 256), lambda b, t: (0, 0)),
            pl.BlockSpec((128, 256), lambda b, t: (0, 0)),
            pl.BlockSpec((128, 256), lambda b, t: (0, 0)),
            pl.BlockSpec((64, 256), lambda b, t: (0, 0)),
            pl.BlockSpec((256,), lambda b, t: (0,)),
            pl.BlockSpec((256, 1), lambda b, t: (0, 0)),
            pl.BlockSpec((1,), lambda b, t: (0,)),
        ],
        out_specs=pl.BlockSpec((1, 1, 256), lambda b, t: (b, 0, t)),
        out_shape=jax.ShapeDtypeStruct((B, 1, Q), jnp.float32),
    )(knn_feat, knn_pos, pe, gmax, lc, vf,
      WposT, bpos, WqT, WkT, WvT, Wgc, Wlc, Wqf, Wpe, Wvf, bd1, Wd2T, bd2)


_THR_PAD = 2e-3
RPW = ROWS // 32
BR = 8
NB = RPW // BR


def _lane(v, j):
    return lax.squeeze(lax.slice(v, (j,), (j + 1,)), (0,))


def _k4_body(gmin_hbm, qpl_hbm, planar_hbm, pf_hbm, kf_out, pos_out,
             xs, ys, zs, qbuf, gm, gidbuf, dbuf, ibuf,
             idxb0, idxb1, fb0, fb1, pb0, pb1,
             semm0, semm1, semg0, semg1, semw0, semw1, semp0, semp1):
    nc = 2
    wid = lax.axis_index("s") * nc + lax.axis_index("c")
    base = wid * RPW
    b = base // Q
    qoff = base % Q
    iota = lax.iota(jnp.int32, 16)
    inf = jnp.float32(jnp.inf)
    zero16 = jnp.zeros((16,), jnp.int32)
    i4 = iota * 4

    pltpu.sync_copy(planar_hbm.at[b, 0], xs)
    pltpu.sync_copy(planar_hbm.at[b, 1], ys)
    pltpu.sync_copy(planar_hbm.at[b, 2], zs)
    pltpu.sync_copy(qpl_hbm.at[b, :, pl.ds(qoff, RPW)], qbuf)

    def zb(j, _):
        o = pl.ds(pl.multiple_of(j * 16, 16), 16)
        pb0[o] = jnp.zeros((16,), jnp.float32)
        pb1[o] = jnp.zeros((16,), jnp.float32)
        return 0
    lax.fori_loop(0, BR * 8, zb, 0)

    def parity(p, fn0, fn1):
        @pl.when(p == 0)
        def _():
            fn0()

        @pl.when(p == 1)
        def _():
            fn1()

    pltpu.async_copy(gmin_hbm.at[pl.ds(base, BR)], gm.at[pl.ds(0, BR)], semm0)

    def block_body(k, _):
        p = k % 2
        r0 = base + k * BR

        parity(p,
               lambda: pltpu.make_async_copy(gmin_hbm.at[pl.ds(r0, BR)],
                                             gm.at[pl.ds(0, BR)],
                                             semm0).wait(),
               lambda: pltpu.make_async_copy(gmin_hbm.at[pl.ds(r0, BR)],
                                             gm.at[pl.ds(BR, BR)],
                                             semm1).wait())

        @pl.when(k + 1 < NB)
        def _():
            rn = r0 + BR
            parity(1 - p,
                   lambda: pltpu.async_copy(gmin_hbm.at[pl.ds(rn, BR)],
                                            gm.at[pl.ds(0, BR)], semm0),
                   lambda: pltpu.async_copy(gmin_hbm.at[pl.ds(rn, BR)],
                                            gm.at[pl.ds(BR, BR)], semm1))

        @pl.when(k >= 2)
        def _():
            parity(p,
                   lambda: (pltpu.make_async_copy(
                       fb0, kf_out.at[pl.ds((r0 - 2 * BR) * K, BR * K)],
                       semw0).wait(),
                       pltpu.make_async_copy(
                       pb0, pos_out.at[pl.ds((r0 - 2 * BR) * 128, BR * 128)],
                       semp0).wait()),
                   lambda: (pltpu.make_async_copy(
                       fb1, kf_out.at[pl.ds((r0 - 2 * BR) * K, BR * K)],
                       semw1).wait(),
                       pltpu.make_async_copy(
                       pb1, pos_out.at[pl.ds((r0 - 2 * BR) * 128, BR * 128)],
                       semp1).wait()))

        def row_body(rb, _):
            i = k * BR + rb
            ri = p * BR + rb
            si = jnp.full((16,), i, jnp.int32)
            qx = plsc.load_gather(qbuf, [zero16, si])
            qy = plsc.load_gather(qbuf, [zero16 + 1, si])
            qz = plsc.load_gather(qbuf, [zero16 + 2, si])

            def gmrow(j):
                return gm[ri, pl.ds(pl.multiple_of(j * 16, 16), 16)]

            def pa(j, acc):
                return jnp.minimum(acc, gmrow(j))
            acc = lax.fori_loop(0, NG // 16, pa,
                                jnp.full((16,), inf, jnp.float32), unroll=4)
            thr = jnp.max(acc) + _THR_PAD
            thrv = thr + (qx * qx + qy * qy + qz * qz)

            def pb_(j, cnt):
                v = gmrow(j)
                m = v <= thr
                pos = cnt + plsc.cumsum(m.astype(jnp.int32)) - 1
                plsc.store_scatter(gidbuf, [pos], iota + j * 16, mask=m)
                return _lane(pos, 15) + 1
            cnt = lax.fori_loop(0, NG // 16, pb_, 0, unroll=4)

            def pc(ci, _c):
                gid = plsc.load_gather(gidbuf,
                                       [jnp.full((16,), ci, jnp.int32)])
                idxv = gid + iota * NG
                px = plsc.load_gather(xs, [idxv])
                py = plsc.load_gather(ys, [idxv])
                pz = plsc.load_gather(zs, [idxv])
                dx, dy, dz = px - qx, py - qy, pz - qz
                d2 = dx * dx + dy * dy + dz * dz
                o = pl.ds(pl.multiple_of(ci * 16, 16), 16)
                dbuf[o] = jnp.where(d2 <= thrv, d2, inf)
                ibuf[o] = idxv
                return 0
            lax.fori_loop(0, cnt, pc, 0)

            def pd(ch, ti):
                tv, tidx = ti
                off = pl.ds(pl.multiple_of(ch * 16, 16), 16)
                dv = dbuf[off]
                hit = plsc.all_reduce_population_count(dv < _lane(tv, 15))

                def merge():
                    iv = ibuf[off]
                    sv, sx = plsc.sort_key_val(dv, iv)
                    rv, ri = lax.rev(sv, (0,)), lax.rev(sx, (0,))
                    keep = tv <= rv
                    sw = plsc.sort_key_val(jnp.where(keep, tv, rv),
                                           jnp.where(keep, tidx, ri))
                    return (sw[0], sw[1])
                return lax.cond(_lane(hit, 0) > 0, merge, lambda: (tv, tidx))
            tv0 = jnp.full((16,), inf, jnp.float32)
            _, topi = lax.fori_loop(0, cnt, pd, (tv0, zero16))

            px = plsc.load_gather(xs, [topi])
            py = plsc.load_gather(ys, [topi])
            pz = plsc.load_gather(zs, [topi])
            pidx = i4 + rb * 128
            io = pl.ds(pl.multiple_of(rb * 16, 16), 16)
            gi = topi + b * N
            parity(p,
                   lambda: (plsc.store_scatter(pb0, [pidx], qx - px),
                            plsc.store_scatter(pb0, [pidx + 1], qy - py),
                            plsc.store_scatter(pb0, [pidx + 2], qz - pz),
                            idxb0.__setitem__(io, gi)),
                   lambda: (plsc.store_scatter(pb1, [pidx], qx - px),
                            plsc.store_scatter(pb1, [pidx + 1], qy - py),
                            plsc.store_scatter(pb1, [pidx + 2], qz - pz),
                            idxb1.__setitem__(io, gi)))
            return 0

        lax.fori_loop(0, BR, row_body, 0)

        parity(p,
               lambda: (pltpu.async_copy(pf_hbm.at[idxb0], fb0, semg0),
                        pltpu.async_copy(
                            pb0, pos_out.at[pl.ds(r0 * 128, BR * 128)],
                            semp0)),
               lambda: (pltpu.async_copy(pf_hbm.at[idxb1], fb1, semg1),
                        pltpu.async_copy(
                            pb1, pos_out.at[pl.ds(r0 * 128, BR * 128)],
                            semp1)))

        @pl.when(k >= 1)
        def _():
            rp = r0 - BR
            parity(1 - p,
                   lambda: (pltpu.make_async_copy(pf_hbm.at[idxb0], fb0,
                                                  semg0).wait(),
                            pltpu.async_copy(
                                fb0, kf_out.at[pl.ds(rp * K, BR * K)], semw0)),
                   lambda: (pltpu.make_async_copy(pf_hbm.at[idxb1], fb1,
                                                  semg1).wait(),
                            pltpu.async_copy(
                                fb1, kf_out.at[pl.ds(rp * K, BR * K)], semw1)))
        return 0

    lax.fori_loop(0, NB, block_body, 0)

    rl = base + (NB - 1) * BR
    pltpu.make_async_copy(pf_hbm.at[idxb1], fb1, semg1).wait()
    pltpu.sync_copy(fb1, kf_out.at[pl.ds(rl * K, BR * K)])
    pltpu.make_async_copy(fb0, kf_out.at[pl.ds((rl - BR) * K, BR * K)],
                          semw0).wait()
    pltpu.make_async_copy(pb0, pos_out.at[pl.ds((rl - BR) * 128, BR * 128)],
                          semp0).wait()
    pltpu.make_async_copy(pb1, pos_out.at[pl.ds(rl * 128, BR * 128)],
                          semp1).wait()


def _run_k4(gmin, qplanar, planar, pf_flat):
    f32, i32 = jnp.float32, jnp.int32
    mesh = plsc.VectorSubcoreMesh(core_axis_name="c", subcore_axis_name="s")
    kfn = functools.partial(
        pl.kernel,
        mesh=mesh,
        compiler_params=pltpu.CompilerParams(needs_layout_passes=False),
        out_type=[
            jax.ShapeDtypeStruct((ROWS * K, 128), f32),
            jax.ShapeDtypeStruct((ROWS * 128,), f32),
        ],
        scratch_types=[
            pltpu.VMEM((N,), f32), pltpu.VMEM((N,), f32), pltpu.VMEM((N,), f32),
            pltpu.VMEM((4, RPW), f32),
            pltpu.VMEM((2 * BR, NG), f32),
            pltpu.VMEM((NG + 16,), i32),
            pltpu.VMEM((N + 16,), f32),
            pltpu.VMEM((N + 16,), i32),
            pltpu.VMEM((BR * K,), i32), pltpu.VMEM((BR * K,), i32),
            pltpu.VMEM((BR * K, 128), f32), pltpu.VMEM((BR * K, 128), f32),
            pltpu.VMEM((BR * 128,), f32), pltpu.VMEM((BR * 128,), f32),
            pltpu.SemaphoreType.DMA, pltpu.SemaphoreType.DMA,
            pltpu.SemaphoreType.DMA, pltpu.SemaphoreType.DMA,
            pltpu.SemaphoreType.DMA, pltpu.SemaphoreType.DMA,
            pltpu.SemaphoreType.DMA, pltpu.SemaphoreType.DMA,
        ],
    )(_k4_body)
    kf, pos = kfn(gmin.reshape(ROWS, NG), qplanar, planar, pf_flat)
    return (kf.reshape(ROWS, K, 128),
            pos.reshape(ROWS, 128)[:, :K * 4].reshape(ROWS, K, 4))


def _knn_placeholder(gmin, query, global_pc, pf):
    d = (jnp.sum(query * query, axis=-1)[..., None]
         - 2.0 * jnp.einsum('bqd,bnd->bqn', query, global_pc)
         + jnp.sum(global_pc * global_pc, axis=-1)[:, None, :])
    _, idx = jax.lax.top_k(-d, K)
    knn_xyz = jax.vmap(lambda p, i: p[i])(global_pc, idx)
    knn_feat = jax.vmap(lambda f, i: f[i])(pf, idx)
    knn_pos = query[:, :, None, :] - knn_xyz
    knn_pos = jnp.concatenate(
        [knn_pos, jnp.zeros(knn_pos.shape[:-1] + (1,), jnp.float32)], axis=-1)
    return (knn_feat.reshape(ROWS, K, 128), knn_pos.reshape(ROWS, K, 4))


def kernel(global_pc, local_pc, query, voxel, Wg1, bg1, Wg2, bg2,
           Wl1, bl1, Wl2, bl2, Wvox, bvox, Wp, bp, gamma, beta,
           Wpos, bpos, Wq, Wk, Wv, Wd1, bd1, Wd2, bd2):
    f32 = jnp.float32
    pad1 = lambda w: jnp.concatenate([w, jnp.zeros((1,) + w.shape[1:], f32)], 0)
    Wg1p = pad1(Wg1)
    Wl1p = pad1(Wl1)
    Wvox2d = jnp.concatenate(
        [Wvox.reshape(64, 27).T, jnp.zeros((5, 64), f32)], axis=0)
    WpT = Wp.T
    Wpp = pad1(WpT)
    WposT = jnp.concatenate([Wpos.T, jnp.zeros((1, 128), f32)], 0)

    pf, gmax, planar = _run_k1(global_pc, Wg1p, bg1, Wg2, bg2)
    gmin = _run_k2(query, planar)
    lc, vf, pe, qplanar = _run_k3(local_pc, voxel, query, Wl1p, bl1, Wl2, bl2,
                                  Wvox2d, bvox, Wpp, bp, gamma, beta)

    knn_feat, knn_pos = _run_k4(gmin, qplanar, planar, pf.reshape(B * N, 128))

    out = _run_k5(knn_feat, knn_pos, pe, gmax, lc, vf,
                  WposT, bpos, Wq.T, Wk.T, Wv.T,
                  Wd1[:, 0:256].T, Wd1[:, 256:512].T, Wd1[:, 512:640].T,
                  Wd1[:, 640:768].T, Wd1[:, 768:832].T, bd1,
                  Wd2.T, bd2)
    return out


# speedup vs baseline: 12.1403x; 1.0001x over previous
import functools
import math

import jax
import jax.numpy as jnp
from jax import lax
from jax.experimental import pallas as pl
from jax.experimental.pallas import tpu as pltpu
from jax.experimental.pallas import tpu_sc as plsc

B, N, M, Q, D, K, C = 4, 8192, 2048, 2048, 32, 16, 128
NG = N // 16
ROWS = B * Q


def _leaky(x):
    return jnp.where(x >= 0, x, 0.2 * x)


def _k1_body(gpc_ref, wg1_ref, bg1_ref, wg2_ref, bg2_ref,
             pf_ref, gmax_ref, planar_ref):
    j = pl.program_id(1)
    g = gpc_ref[0]
    gp = jnp.concatenate([g, jnp.zeros((g.shape[0], 1), jnp.float32)], axis=1)
    pf = _leaky(jax.lax.dot_general(gp, wg1_ref[...],
                                    (((1,), (0,)), ((), ())),
                                    preferred_element_type=jnp.float32)
                + bg1_ref[...][None, :])
    pf_ref[0] = pf
    gg = _leaky(jax.lax.dot_general(pf, wg2_ref[...],
                                    (((1,), (0,)), ((), ())),
                                    preferred_element_type=jnp.float32)
                + bg2_ref[...][None, :])
    part = jnp.max(gg, axis=0, keepdims=True)

    @pl.when(j == 0)
    def _():
        gmax_ref[...] = jnp.full_like(gmax_ref, -jnp.inf)

    gmax_ref[0] = jnp.maximum(gmax_ref[0], part)
    planar_ref[0] = jnp.transpose(gp, (1, 0))


def _run_k1(global_pc, Wg1p, bg1, Wg2, bg2):
    return pl.pallas_call(
        _k1_body,
        grid=(B, N // 1024),
        in_specs=[
            pl.BlockSpec((1, 1024, 3), lambda b, j: (b, j, 0)),
            pl.BlockSpec((4, 128), lambda b, j: (0, 0)),
            pl.BlockSpec((128,), lambda b, j: (0,)),
            pl.BlockSpec((128, 256), lambda b, j: (0, 0)),
            pl.BlockSpec((256,), lambda b, j: (0,)),
        ],
        out_specs=[
            pl.BlockSpec((1, 1024, 128), lambda b, j: (b, j, 0)),
            pl.BlockSpec((1, 1, 256), lambda b, j: (b, 0, 0)),
            pl.BlockSpec((1, 4, 1024), lambda b, j: (b, 0, j)),
        ],
        out_shape=[
            jax.ShapeDtypeStruct((B, N, 128), jnp.float32),
            jax.ShapeDtypeStruct((B, 1, 256), jnp.float32),
            jax.ShapeDtypeStruct((B, 4, N), jnp.float32),
        ],
    )(global_pc, Wg1p, bg1, Wg2, bg2)


def _k2_body(q_ref, g_ref, gmin_ref):
    j = pl.program_id(2)
    q = q_ref[0]
    gt = g_ref[0]
    qp = jnp.concatenate([q, jnp.zeros((q.shape[0], 1), jnp.float32)], axis=1)
    gg = jnp.sum(gt * gt, axis=0, keepdims=True)
    cross = jax.lax.dot_general(qp, gt, (((1,), (0,)), ((), ())),
                                preferred_element_type=jnp.float32)
    part = gg - 2.0 * cross

    @pl.when(j == 0)
    def _():
        gmin_ref[0] = jnp.full_like(gmin_ref[0], jnp.inf)

    gmin_ref[0] = jnp.minimum(gmin_ref[0], part)


def _run_k2(query, planar):
    return pl.pallas_call(
        _k2_body,
        grid=(B, Q // 512, N // NG),
        in_specs=[
            pl.BlockSpec((1, 512, 3), lambda b, i, j: (b, i, 0)),
            pl.BlockSpec((1, 4, NG), lambda b, i, j: (b, 0, j)),
        ],
        out_specs=pl.BlockSpec((1, 512, NG), lambda b, i, j: (b, i, 0)),
        out_shape=jax.ShapeDtypeStruct((B, Q, NG), jnp.float32),
    )(query, planar)


def _k3_body(lpc_ref, vox_ref, q_ref,
             wl1_ref, bl1_ref, wl2_ref, bl2_ref,
             wvox_ref, bvox_ref, wp_ref, bp_ref, gamma_ref, beta_ref,
             lc_ref, vf_ref, pe_ref, qplanar_ref):
    lp = lpc_ref[...].reshape(B * M, 3)
    lp4 = jnp.concatenate([lp, jnp.zeros((B * M, 1), jnp.float32)], axis=1)
    lh = _leaky(jax.lax.dot_general(lp4, wl1_ref[...], (((1,), (0,)), ((), ())),
                                    preferred_element_type=jnp.float32)
                + bl1_ref[...][None, :])
    l2 = _leaky(jax.lax.dot_general(lh, wl2_ref[...], (((1,), (0,)), ((), ())),
                                    preferred_element_type=jnp.float32)
                + bl2_ref[...][None, :])
    lc_ref[...] = jnp.max(l2.reshape(B, M, 256), axis=1, keepdims=True)

    w2d = wvox_ref[...]
    for b in range(B):
        conv = jax.lax.dot_general(w2d, vox_ref[b], (((0,), (0,)), ((), ())),
                                   preferred_element_type=jnp.float32)
        conv = _leaky(conv + bvox_ref[...][:, None])
        vf_ref[b, 0] = jnp.mean(conv, axis=1)

    q = q_ref[...].reshape(B * Q, 3)
    q4 = jnp.concatenate([q, jnp.zeros((B * Q, 1), jnp.float32)], axis=1)
    pe_raw = jax.lax.dot_general(q4, wp_ref[...], (((1,), (0,)), ((), ())),
                                 preferred_element_type=jnp.float32) \
        + bp_ref[...][None, :]
    qbar = jnp.mean(q4, axis=0, keepdims=True)
    second = jax.lax.dot_general(q4, q4, (((0,), (0,)), ((), ())),
                                 preferred_element_type=jnp.float32) / (B * Q)
    cov = second - jax.lax.dot_general(qbar, qbar, (((0,), (0,)), ((), ())),
                                       preferred_element_type=jnp.float32)
    wp = wp_ref[...]
    wc = jax.lax.dot_general(cov, wp, (((1,), (0,)), ((), ())),
                             preferred_element_type=jnp.float32)
    var = jnp.sum(wp * wc, axis=0)
    mu = jax.lax.dot_general(qbar, wp, (((1,), (0,)), ((), ())),
                             preferred_element_type=jnp.float32)[0] + bp_ref[...]
    inv = gamma_ref[...] / jnp.sqrt(var + 1e-5)
    pe = _leaky((pe_raw - mu[None, :]) * inv[None, :] + beta_ref[...][None, :])
    pe_ref[...] = pe.reshape(B, Q, 128)
    qplanar_ref[...] = jnp.transpose(q4.reshape(B, Q, 4), (0, 2, 1))


def _voxel_patches(voxel):
    v = voxel[..., 0]
    vp = jnp.pad(v, ((0, 0), (0, 2), (0, 2), (0, 2)))
    cols = []
    for i in range(3):
        for j in range(3):
            for k in range(3):
                c = lax.slice(vp, (0, i, j, k), (B, i + 32, j + 32, k + 32),
                              (1, 2, 2, 2))
                cols.append(c.reshape(B, 1, 4096))
    cols.append(jnp.zeros((B, 5, 4096), jnp.float32))
    return jnp.concatenate(cols, axis=1)


def _run_k3(local_pc, voxel, query, Wl1p, bl1, Wl2, bl2, Wvox2d, bvox,
            Wpp, bp, gamma, beta):
    voxel = _voxel_patches(voxel)
    return pl.pallas_call(
        _k3_body,
        out_shape=[
            jax.ShapeDtypeStruct((B, 1, 256), jnp.float32),
            jax.ShapeDtypeStruct((B, 1, 64), jnp.float32),
            jax.ShapeDtypeStruct((B, Q, 128), jnp.float32),
            jax.ShapeDtypeStruct((B, 4, Q), jnp.float32),
        ],
    )(local_pc, voxel, query, Wl1p, bl1, Wl2, bl2, Wvox2d, bvox,
      Wpp, bp, gamma, beta)


def _k5_body(kf_ref, pos_ref, pe_ref, gc_ref, lc_ref, vf_ref,
             wpos_ref, bpos_ref, wq_ref, wk_ref, wv_ref,
             wgc_ref, wlc_ref, wqf_ref, wpe_ref, wvf_ref, bd1_ref,
             wd2_ref, bd2_ref, out_ref):
    kf = kf_ref[...].reshape(256 * K, 128)
    pos = pos_ref[...].reshape(256 * K, 4)
    pos_enc = _leaky(jax.lax.dot_general(pos, wpos_ref[...],
                                         (((1,), (0,)), ((), ())),
                                         preferred_element_type=jnp.float32)
                     + bpos_ref[...][None, :])
    fq = jax.lax.dot_general(kf, wq_ref[...], (((1,), (0,)), ((), ())),
                             preferred_element_type=jnp.float32)
    fk = jax.lax.dot_general(pos_enc, wk_ref[...], (((1,), (0,)), ((), ())),
                             preferred_element_type=jnp.float32)
    fv = jax.lax.dot_general(kf + pos_enc, wv_ref[...], (((1,), (0,)), ((), ())),
                             preferred_element_type=jnp.float32)
    logits = jnp.sum(fq * fk, axis=1).reshape(256, K) / math.sqrt(float(C))
    mx = jnp.max(logits, axis=1, keepdims=True)
    e = jnp.exp(logits - mx)
    attn = e / jnp.sum(e, axis=1, keepdims=True)
    qf = jnp.sum(attn[:, :, None] * fv.reshape(256, K, 128), axis=1)

    cbase = (jax.lax.dot_general(gc_ref[0], wgc_ref[...],
                                 (((1,), (0,)), ((), ())),
                                 preferred_element_type=jnp.float32)
             + jax.lax.dot_general(lc_ref[0], wlc_ref[...],
                                   (((1,), (0,)), ((), ())),
                                   preferred_element_type=jnp.float32)
             + jax.lax.dot_general(vf_ref[0], wvf_ref[...],
                                   (((1,), (0,)), ((), ())),
                                   preferred_element_type=jnp.float32))
    pe = pe_ref[0]
    h = _leaky(jax.lax.dot_general(qf, wqf_ref[...], (((1,), (0,)), ((), ())),
                                   preferred_element_type=jnp.float32)
               + jax.lax.dot_general(pe, wpe_ref[...], (((1,), (0,)), ((), ())),
                                     preferred_element_type=jnp.float32)
               + cbase + bd1_ref[...][None, :])
    o = jax.lax.dot_general(h, wd2_ref[...], (((1,), (0,)), ((), ())),
                            preferred_element_type=jnp.float32) + bd2_ref[...]
    out_ref[0, 0] = o[:, 0]


def _run_k5(knn_feat, knn_pos, pe, gmax, lc, vf,
            WposT, bpos, WqT, WkT, WvT, Wgc, Wlc, Wqf, Wpe, Wvf, bd1,
            Wd2T, bd2):
    nt = Q // 256
    return pl.pallas_call(
        _k5_body,
        grid=(B, nt),
        in_specs=[
            pl.BlockSpec((256, K, 128), lambda b, t: (b * nt + t, 0, 0)),
            pl.BlockSpec((256, K, 4), lambda b, t: (b * nt + t, 0, 0)),
            pl.BlockSpec((1, 256, 128), lambda b, t: (b, t, 0)),
            pl.BlockSpec((1, 1, 256), lambda b, t: (b, 0, 0)),
            pl.BlockSpec((1, 1, 256), lambda b, t: (b, 0, 0)),
            pl.BlockSpec((1, 1, 64), lambda b, t: (b, 0, 0)),
            pl.BlockSpec((4, 128), lambda b, t: (0, 0)),
            pl.BlockSpec((128,), lambda b, t: (0,)),
            pl.BlockSpec((128, 128), lambda b, t: (0, 0)),
            pl.BlockSpec((128, 128), lambda b, t: (0, 0)),
            pl.BlockSpec((128, 128), lambda b, t: (0, 0)),
            pl.BlockSpec((256, 256), lambda b, t: (0, 0)),
            pl.BlockSpec((256, 256), lambda b, t: (0, 0)),
            pl.BlockSpec((128, 256), lambda b, t: (0, 0)),
            pl.BlockSpec((128, 256), lambda b, t: (0, 0)),
            pl.BlockSpec((64, 256), lambda b, t: (0, 0)),
            pl.BlockSpec((256,), lambda b, t: (0,)),
            pl.BlockSpec((256, 1), lambda b, t: (0, 0)),
            pl.BlockSpec((1,), lambda b, t: (0,)),
        ],
        out_specs=pl.BlockSpec((1, 1, 256), lambda b, t: (b, 0, t)),
        out_shape=jax.ShapeDtypeStruct((B, 1, Q), jnp.float32),
    )(knn_feat, knn_pos, pe, gmax, lc, vf,
      WposT, bpos, WqT, WkT, WvT, Wgc, Wlc, Wqf, Wpe, Wvf, bd1, Wd2T, bd2)


_THR_PAD = 2e-3
RPW = ROWS // 32
BR = 8
NB = RPW // BR


def _lane(v, j):
    return lax.squeeze(lax.slice(v, (j,), (j + 1,)), (0,))


def _k4_body(gmin_hbm, qpl_hbm, planar_hbm, pf_hbm, kf_out, pos_out,
             xs, ys, zs, qbuf, gm, gidbuf, dbuf, ibuf,
             idxb0, idxb1, fb0, fb1, pb0, pb1,
             semm0, semm1, semg0, semg1, semw0, semw1, semp0, semp1):
    nc = 2
    wid = lax.axis_index("s") * nc + lax.axis_index("c")
    base = wid * RPW
    b = base // Q
    qoff = base % Q
    iota = lax.iota(jnp.int32, 16)
    inf = jnp.float32(jnp.inf)
    zero16 = jnp.zeros((16,), jnp.int32)
    i4 = iota * 4

    pltpu.sync_copy(planar_hbm.at[b, 0], xs)
    pltpu.sync_copy(planar_hbm.at[b, 1], ys)
    pltpu.sync_copy(planar_hbm.at[b, 2], zs)
    pltpu.sync_copy(qpl_hbm.at[b, :, pl.ds(qoff, RPW)], qbuf)

    def zb(j, _):
        o = pl.ds(pl.multiple_of(j * 16, 16), 16)
        pb0[o] = jnp.zeros((16,), jnp.float32)
        pb1[o] = jnp.zeros((16,), jnp.float32)
        return 0
    lax.fori_loop(0, BR * 8, zb, 0)

    def parity(p, fn0, fn1):
        @pl.when(p == 0)
        def _():
            fn0()

        @pl.when(p == 1)
        def _():
            fn1()

    pltpu.async_copy(gmin_hbm.at[pl.ds(base, BR)], gm.at[pl.ds(0, BR)], semm0)

    def block_body(k, _):
        p = k % 2
        r0 = base + k * BR

        parity(p,
               lambda: pltpu.make_async_copy(gmin_hbm.at[pl.ds(r0, BR)],
                                             gm.at[pl.ds(0, BR)],
                                             semm0).wait(),
               lambda: pltpu.make_async_copy(gmin_hbm.at[pl.ds(r0, BR)],
                                             gm.at[pl.ds(BR, BR)],
                                             semm1).wait())

        @pl.when(k + 1 < NB)
        def _():
            rn = r0 + BR
            parity(1 - p,
                   lambda: pltpu.async_copy(gmin_hbm.at[pl.ds(rn, BR)],
                                            gm.at[pl.ds(0, BR)], semm0),
                   lambda: pltpu.async_copy(gmin_hbm.at[pl.ds(rn, BR)],
                                            gm.at[pl.ds(BR, BR)], semm1))

        @pl.when(k >= 2)
        def _():
            parity(p,
                   lambda: (pltpu.make_async_copy(
                       fb0, kf_out.at[pl.ds((r0 - 2 * BR) * K, BR * K)],
                       semw0).wait(),
                       pltpu.make_async_copy(
                       pb0, pos_out.at[pl.ds((r0 - 2 * BR) * 128, BR * 128)],
                       semp0).wait()),
                   lambda: (pltpu.make_async_copy(
                       fb1, kf_out.at[pl.ds((r0 - 2 * BR) * K, BR * K)],
                       semw1).wait(),
                       pltpu.make_async_copy(
                       pb1, pos_out.at[pl.ds((r0 - 2 * BR) * 128, BR * 128)],
                       semp1).wait()))

        def row_body(rb, _):
            i = k * BR + rb
            ri = p * BR + rb
            si = jnp.full((16,), i, jnp.int32)
            qx = plsc.load_gather(qbuf, [zero16, si])
            qy = plsc.load_gather(qbuf, [zero16 + 1, si])
            qz = plsc.load_gather(qbuf, [zero16 + 2, si])

            def gmrow(j):
                return gm[ri, pl.ds(pl.multiple_of(j * 16, 16), 16)]

            def pa(j, acc):
                return jnp.minimum(acc, gmrow(j))
            acc = lax.fori_loop(0, NG // 16, pa,
                                jnp.full((16,), inf, jnp.float32), unroll=4)
            thr = jnp.max(acc) + _THR_PAD
            thrv = thr + (qx * qx + qy * qy + qz * qz)

            def pb_(j, cnt):
                v = gmrow(j)
                m = v <= thr
                pos = cnt + plsc.cumsum(m.astype(jnp.int32)) - 1
                plsc.store_scatter(gidbuf, [pos], iota + j * 16, mask=m)
                return _lane(pos, 15) + 1
            cnt = lax.fori_loop(0, NG // 16, pb_, 0, unroll=4)

            def pc(ci, _c):
                gid = plsc.load_gather(gidbuf,
                                       [jnp.full((16,), ci, jnp.int32)])
                addr = gid * 16 + iota
                idxv = gid + iota * NG
                px = plsc.load_gather(xs, [addr])
                py = plsc.load_gather(ys, [addr])
                pz = plsc.load_gather(zs, [addr])
                dx, dy, dz = px - qx, py - qy, pz - qz
                d2 = dx * dx + dy * dy + dz * dz
                o = pl.ds(pl.multiple_of(ci * 16, 16), 16)
                dbuf[o] = jnp.where(d2 <= thrv, d2, inf)
                ibuf[o] = idxv
                return 0
            lax.fori_loop(0, cnt, pc, 0)

            def pd(ch, ti):
                tv, tidx = ti
                off = pl.ds(pl.multiple_of(ch * 16, 16), 16)
                dv = dbuf[off]
                hit = plsc.all_reduce_population_count(dv < _lane(tv, 15))

                def merge():
                    iv = ibuf[off]
                    sv, sx = plsc.sort_key_val(dv, iv)
                    rv, ri = lax.rev(sv, (0,)), lax.rev(sx, (0,))
                    keep = tv <= rv
                    sw = plsc.sort_key_val(jnp.where(keep, tv, rv),
                                           jnp.where(keep, tidx, ri))
                    return (sw[0], sw[1])
                return lax.cond(_lane(hit, 0) > 0, merge, lambda: (tv, tidx))
            tv0 = jnp.full((16,), inf, jnp.float32)
            _, topi = lax.fori_loop(0, cnt, pd, (tv0, zero16))

            taddr = ((topi & (NG - 1)) * 16) + (topi >> 9)
            px = plsc.load_gather(xs, [taddr])
            py = plsc.load_gather(ys, [taddr])
            pz = plsc.load_gather(zs, [taddr])
            pidx = i4 + rb * 128
            io = pl.ds(pl.multiple_of(rb * 16, 16), 16)
            gi = topi + b * N
            parity(p,
                   lambda: (plsc.store_scatter(pb0, [pidx], qx - px),
                            plsc.store_scatter(pb0, [pidx + 1], qy - py),
                            plsc.store_scatter(pb0, [pidx + 2], qz - pz),
                            idxb0.__setitem__(io, gi)),
                   lambda: (plsc.store_scatter(pb1, [pidx], qx - px),
                            plsc.store_scatter(pb1, [pidx + 1], qy - py),
                            plsc.store_scatter(pb1, [pidx + 2], qz - pz),
                            idxb1.__setitem__(io, gi)))
            return 0

        lax.fori_loop(0, BR, row_body, 0)

        parity(p,
               lambda: (pltpu.async_copy(pf_hbm.at[idxb0], fb0, semg0),
                        pltpu.async_copy(
                            pb0, pos_out.at[pl.ds(r0 * 128, BR * 128)],
                            semp0)),
               lambda: (pltpu.async_copy(pf_hbm.at[idxb1], fb1, semg1),
                        pltpu.async_copy(
                            pb1, pos_out.at[pl.ds(r0 * 128, BR * 128)],
                            semp1)))

        @pl.when(k >= 1)
        def _():
            rp = r0 - BR
            parity(1 - p,
                   lambda: (pltpu.make_async_copy(pf_hbm.at[idxb0], fb0,
                                                  semg0).wait(),
                            pltpu.async_copy(
                                fb0, kf_out.at[pl.ds(rp * K, BR * K)], semw0)),
                   lambda: (pltpu.make_async_copy(pf_hbm.at[idxb1], fb1,
                                                  semg1).wait(),
                            pltpu.async_copy(
                                fb1, kf_out.at[pl.ds(rp * K, BR * K)], semw1)))
        return 0

    lax.fori_loop(0, NB, block_body, 0)

    rl = base + (NB - 1) * BR
    pltpu.make_async_copy(pf_hbm.at[idxb1], fb1, semg1).wait()
    pltpu.sync_copy(fb1, kf_out.at[pl.ds(rl * K, BR * K)])
    pltpu.make_async_copy(fb0, kf_out.at[pl.ds((rl - BR) * K, BR * K)],
                          semw0).wait()
    pltpu.make_async_copy(pb0, pos_out.at[pl.ds((rl - BR) * 128, BR * 128)],
                          semp0).wait()
    pltpu.make_async_copy(pb1, pos_out.at[pl.ds(rl * 128, BR * 128)],
                          semp1).wait()


def _run_k4(gmin, qplanar, planar, pf_flat):
    f32, i32 = jnp.float32, jnp.int32
    mesh = plsc.VectorSubcoreMesh(core_axis_name="c", subcore_axis_name="s")
    kfn = functools.partial(
        pl.kernel,
        mesh=mesh,
        compiler_params=pltpu.CompilerParams(needs_layout_passes=False),
        out_type=[
            jax.ShapeDtypeStruct((ROWS * K, 128), f32),
            jax.ShapeDtypeStruct((ROWS * 128,), f32),
        ],
        scratch_types=[
            pltpu.VMEM((N,), f32), pltpu.VMEM((N,), f32), pltpu.VMEM((N,), f32),
            pltpu.VMEM((4, RPW), f32),
            pltpu.VMEM((2 * BR, NG), f32),
            pltpu.VMEM((NG + 16,), i32),
            pltpu.VMEM((N + 16,), f32),
            pltpu.VMEM((N + 16,), i32),
            pltpu.VMEM((BR * K,), i32), pltpu.VMEM((BR * K,), i32),
            pltpu.VMEM((BR * K, 128), f32), pltpu.VMEM((BR * K, 128), f32),
            pltpu.VMEM((BR * 128,), f32), pltpu.VMEM((BR * 128,), f32),
            pltpu.SemaphoreType.DMA, pltpu.SemaphoreType.DMA,
            pltpu.SemaphoreType.DMA, pltpu.SemaphoreType.DMA,
            pltpu.SemaphoreType.DMA, pltpu.SemaphoreType.DMA,
            pltpu.SemaphoreType.DMA, pltpu.SemaphoreType.DMA,
        ],
    )(_k4_body)
    planar_shuf = jnp.transpose(planar.reshape(B, 4, 16, NG),
                                (0, 1, 3, 2)).reshape(B, 4, N)
    kf, pos = kfn(gmin.reshape(ROWS, NG), qplanar, planar_shuf, pf_flat)
    return (kf.reshape(ROWS, K, 128),
            pos.reshape(ROWS, 128)[:, :K * 4].reshape(ROWS, K, 4))


def _knn_placeholder(gmin, query, global_pc, pf):
    d = (jnp.sum(query * query, axis=-1)[..., None]
         - 2.0 * jnp.einsum('bqd,bnd->bqn', query, global_pc)
         + jnp.sum(global_pc * global_pc, axis=-1)[:, None, :])
    _, idx = jax.lax.top_k(-d, K)
    knn_xyz = jax.vmap(lambda p, i: p[i])(global_pc, idx)
    knn_feat = jax.vmap(lambda f, i: f[i])(pf, idx)
    knn_pos = query[:, :, None, :] - knn_xyz
    knn_pos = jnp.concatenate(
        [knn_pos, jnp.zeros(knn_pos.shape[:-1] + (1,), jnp.float32)], axis=-1)
    return (knn_feat.reshape(ROWS, K, 128), knn_pos.reshape(ROWS, K, 4))


def kernel(global_pc, local_pc, query, voxel, Wg1, bg1, Wg2, bg2,
           Wl1, bl1, Wl2, bl2, Wvox, bvox, Wp, bp, gamma, beta,
           Wpos, bpos, Wq, Wk, Wv, Wd1, bd1, Wd2, bd2):
    f32 = jnp.float32
    pad1 = lambda w: jnp.concatenate([w, jnp.zeros((1,) + w.shape[1:], f32)], 0)
    Wg1p = pad1(Wg1)
    Wl1p = pad1(Wl1)
    Wvox2d = jnp.concatenate(
        [Wvox.reshape(64, 27).T, jnp.zeros((5, 64), f32)], axis=0)
    WpT = Wp.T
    Wpp = pad1(WpT)
    WposT = jnp.concatenate([Wpos.T, jnp.zeros((1, 128), f32)], 0)

    pf, gmax, planar = _run_k1(global_pc, Wg1p, bg1, Wg2, bg2)
    gmin = _run_k2(query, planar)
    lc, vf, pe, qplanar = _run_k3(local_pc, voxel, query, Wl1p, bl1, Wl2, bl2,
                                  Wvox2d, bvox, Wpp, bp, gamma, beta)

    knn_feat, knn_pos = _run_k4(gmin, qplanar, planar, pf.reshape(B * N, 128))

    out = _run_k5(knn_feat, knn_pos, pe, gmax, lc, vf,
                  WposT, bpos, Wq.T, Wk.T, Wv.T,
                  Wd1[:, 0:256].T, Wd1[:, 256:512].T, Wd1[:, 512:640].T,
                  Wd1[:, 640:768].T, Wd1[:, 768:832].T, bd1,
                  Wd2.T, bd2)
    return out
